# Initial kernel scaffold; baseline (speedup 1.0000x reference)
import jax
import jax.numpy as jnp
from jax.experimental import pallas as pl


def kernel(x, edge_index, conv1_w, conv1_b, conv2_w, conv2_b, W1, b1, W2, b2, fc_W, fc_b):
    raise NotImplementedError("write your pallas kernel here")


# speedup vs baseline: 13.2530x; 13.2530x over previous
import functools

import jax
import jax.numpy as jnp
from jax import lax
from jax.experimental import pallas as pl
from jax.experimental.pallas import tpu as pltpu
from jax.experimental.pallas import tpu_sc as plsc

N = 50000
NPAD = 51200
E = 800000
EPAD = 819200
ECH = EPAD // 128
CPW32 = ECH // 32
CPW16 = ECH // 16
RPS = NPAD // 16


def _mm(a, b):
    return jax.lax.dot_general(
        a, b, (((1,), (0,)), ((), ())),
        precision=jax.lax.Precision.HIGHEST,
        preferred_element_type=jnp.float32)


def _k1_body(xq_ref, wA_ref, wB_ref, w2s_ref, b1_ref, b2_ref, out_ref,
             p1e_ref, p1o_ref):
    xq = xq_ref[0]
    x0 = xq[0:500]
    x1 = xq[1:501]
    b1v = b1_ref[...]
    c = [jnp.maximum(_mm(x0, wA_ref[p]) + _mm(x1, wB_ref[p]) + b1v, 0.0)
         for p in range(4)]
    p1e_ref[0:1, :] = jnp.zeros((1, 16), jnp.float32)
    p1e_ref[1:501, :] = jnp.maximum(c[0], c[1])
    p1e_ref[501:502, :] = jnp.zeros((1, 16), jnp.float32)
    p1o_ref[0:1, :] = jnp.zeros((1, 16), jnp.float32)
    p1o_ref[1:501, :] = jnp.maximum(c[2], c[3])
    p1o_ref[501:502, :] = jnp.zeros((1, 16), jnp.float32)
    e_m1 = p1e_ref[0:500]
    e_0 = p1e_ref[1:501]
    e_p1 = p1e_ref[2:502]
    o_m1 = p1o_ref[0:500]
    o_0 = p1o_ref[1:501]
    o_p1 = p1o_ref[2:502]
    b2v = b2_ref[...]
    c2e = (_mm(e_m1, w2s_ref[0]) + _mm(o_m1, w2s_ref[1]) + _mm(e_0, w2s_ref[2])
           + _mm(o_0, w2s_ref[3]) + _mm(e_p1, w2s_ref[4]) + b2v)
    c2o = (_mm(o_m1, w2s_ref[0]) + _mm(e_0, w2s_ref[1]) + _mm(o_0, w2s_ref[2])
           + _mm(e_p1, w2s_ref[3]) + _mm(o_p1, w2s_ref[4]) + b2v)
    out_ref[0] = jnp.maximum(jnp.maximum(c2e, 0.0), jnp.maximum(c2o, 0.0))


def _conv_stack(xq, wA, wB, w2s, b1r, b2r):
    return pl.pallas_call(
        _k1_body,
        grid=(100,),
        in_specs=[
            pl.BlockSpec((1, 501, 36), lambda i: (i, 0, 0)),
            pl.BlockSpec((4, 36, 16), lambda i: (0, 0, 0)),
            pl.BlockSpec((4, 36, 16), lambda i: (0, 0, 0)),
            pl.BlockSpec((5, 16, 32), lambda i: (0, 0, 0)),
            pl.BlockSpec((1, 16), lambda i: (0, 0)),
            pl.BlockSpec((1, 32), lambda i: (0, 0)),
        ],
        out_specs=pl.BlockSpec((1, 500, 32), lambda i: (i, 0, 0)),
        out_shape=jax.ShapeDtypeStruct((100, 500, 32), jnp.float32),
        scratch_shapes=[pltpu.VMEM((502, 16), jnp.float32),
                        pltpu.VMEM((502, 16), jnp.float32)],
    )(xq, wA, wB, w2s, b1r, b2r)


def _sc_degree(dstp):
    mesh = plsc.VectorSubcoreMesh(core_axis_name="c", subcore_axis_name="s")

    @functools.partial(
        pl.kernel,
        out_type=jax.ShapeDtypeStruct((2, 1, NPAD), jnp.float32),
        mesh=mesh,
        compiler_params=pltpu.CompilerParams(use_tc_tiling_on_sc=False),
        scratch_types=[
            pltpu.VMEM((CPW32, 128), jnp.int32),
            pltpu.VMEM((128,), jnp.float32),
            pltpu.VMEM((RPS,), jnp.float32),
            pltpu.VMEM_SHARED((NPAD,), jnp.float32),
        ])
    def deg_k(dst_hbm, out_hbm, idx_v, ones_v, zs_v, acc_sh):
        cid = lax.axis_index("c")
        sid = lax.axis_index("s")
        w = cid * 16 + sid

        def fill_ones(i, _):
            ones_v[pl.ds(i * 16, 16)] = jnp.ones((16,), jnp.float32)
            return 0
        lax.fori_loop(0, 8, fill_ones, 0)

        def fill_z(i, _):
            zs_v[pl.ds(i * 16, 16)] = jnp.zeros((16,), jnp.float32)
            return 0
        lax.fori_loop(0, RPS // 16, fill_z, 0)

        pltpu.sync_copy(zs_v, acc_sh.at[pl.ds(sid * RPS, RPS)])
        plsc.subcore_barrier()
        pltpu.sync_copy(dst_hbm.at[pl.ds(w * CPW32, CPW32)], idx_v)

        def body(j, _):
            pltpu.sync_copy(ones_v, acc_sh.at[idx_v.at[j]], add=True)
            return 0
        lax.fori_loop(0, CPW32, body, 0)

        plsc.subcore_barrier()
        pltpu.sync_copy(acc_sh.at[pl.ds(sid * RPS, RPS)],
                        out_hbm.at[cid, 0, pl.ds(sid * RPS, RPS)])

    return deg_k(dstp)


def _sc_agg(tab, srcp, dstp, feat_split):
    mesh = plsc.VectorSubcoreMesh(core_axis_name="c", subcore_axis_name="s")
    cpw = CPW16 if feat_split else CPW32
    TCH = 25
    ntiles = cpw // TCH

    @functools.partial(
        pl.kernel,
        out_type=jax.ShapeDtypeStruct((2, NPAD, 32), jnp.float32),
        mesh=mesh,
        compiler_params=pltpu.CompilerParams(use_tc_tiling_on_sc=False),
        scratch_types=[
            pltpu.VMEM((TCH, 128), jnp.int32),
            pltpu.VMEM((TCH, 128), jnp.int32),
            pltpu.VMEM((128, 32), jnp.float32),
            pltpu.VMEM((100, 32), jnp.float32),
            pltpu.VMEM_SHARED((NPAD, 32), jnp.float32),
            pltpu.SemaphoreType.DMA,
        ])
    def agg_k(tab_hbm, src_hbm, dst_hbm, out_hbm, srcv, dstv, rows_v, zr_v,
              acc_sh, sem):
        cid = lax.axis_index("c")
        sid = lax.axis_index("s")

        def fill_z(i, _):
            zr_v[i, pl.ds(0, 16)] = jnp.zeros((16,), jnp.float32)
            zr_v[i, pl.ds(16, 16)] = jnp.zeros((16,), jnp.float32)
            return 0
        lax.fori_loop(0, 100, fill_z, 0)

        def zero_acc(k, _):
            pltpu.sync_copy(zr_v, acc_sh.at[pl.ds(sid * RPS + k * 100, 100)])
            return 0
        lax.fori_loop(0, 32, zero_acc, 0)
        plsc.subcore_barrier()

        def tile_body(t, _):
            if feat_split:
                base = sid * cpw + t * TCH
                pltpu.sync_copy(src_hbm.at[cid, pl.ds(base, TCH)], srcv)
                pltpu.sync_copy(dst_hbm.at[pl.ds(base, TCH)], dstv)
            else:
                base = (cid * 16 + sid) * cpw + t * TCH
                pltpu.sync_copy(src_hbm.at[pl.ds(base, TCH)], srcv)
                pltpu.sync_copy(dst_hbm.at[pl.ds(base, TCH)], dstv)

            def body(j, _):
                pltpu.async_copy(tab_hbm.at[srcv.at[j]], rows_v, sem).wait()
                pltpu.sync_copy(rows_v, acc_sh.at[dstv.at[j]], add=True)
                return 0
            lax.fori_loop(0, TCH, body, 0)
            return 0
        lax.fori_loop(0, ntiles, tile_body, 0)

        plsc.subcore_barrier()
        pltpu.sync_copy(acc_sh.at[pl.ds(sid * RPS, RPS)],
                        out_hbm.at[cid, pl.ds(sid * RPS, RPS)])

    return agg_k(tab, srcp, dstp)


def _k2_body(degt_ref, h0_ref, dinv_ref, hs1_ref):
    d = degt_ref[:, 0:1] + degt_ref[:, 1:2] + 1.0
    dv = lax.rsqrt(d)
    dinv_ref[...] = dv
    hs1_ref[...] = h0_ref[...] * dv


def _scale_kernel(deg2t, h0p):
    return pl.pallas_call(
        _k2_body,
        grid=(64,),
        in_specs=[
            pl.BlockSpec((800, 2), lambda i: (i, 0)),
            pl.BlockSpec((800, 32), lambda i: (i, 0)),
        ],
        out_specs=[
            pl.BlockSpec((800, 1), lambda i: (i, 0)),
            pl.BlockSpec((800, 32), lambda i: (i, 0)),
        ],
        out_shape=[jax.ShapeDtypeStruct((NPAD, 1), jnp.float32),
                   jax.ShapeDtypeStruct((NPAD, 32), jnp.float32)],
    )(deg2t, h0p)


def _k3_body(agg_ref, hs1_ref, dinv_ref, W1_ref, b1_ref, out_ref):
    i = pl.program_id(0)
    dv = dinv_ref[...]
    t1 = (agg_ref[0] + agg_ref[1] + hs1_ref[...]) * dv
    h1 = jnp.maximum(_mm(t1, W1_ref[...]) + b1_ref[...], 0.0)
    rows = i * 800 + lax.broadcasted_iota(jnp.int32, (800, 1), 0)
    hs2 = jnp.where(rows < N, h1 * dv, 0.0)
    out_ref[0] = hs2[:, 0:32]
    out_ref[1] = hs2[:, 32:64]


def _h1_kernel(agg1, hs1, dinv, W1, b1r):
    return pl.pallas_call(
        _k3_body,
        grid=(64,),
        in_specs=[
            pl.BlockSpec((2, 800, 32), lambda i: (0, i, 0)),
            pl.BlockSpec((800, 32), lambda i: (i, 0)),
            pl.BlockSpec((800, 1), lambda i: (i, 0)),
            pl.BlockSpec((32, 64), lambda i: (0, 0)),
            pl.BlockSpec((1, 64), lambda i: (0, 0)),
        ],
        out_specs=pl.BlockSpec((2, 800, 32), lambda i: (0, i, 0)),
        out_shape=jax.ShapeDtypeStruct((2, NPAD, 32), jnp.float32),
    )(agg1, hs1, dinv, W1, b1r)


def _k4_body(agg_ref, hs_ref, dinv_ref, W2_ref, b2_ref, fcW_ref, fcb_ref,
             out_ref):
    i = pl.program_id(0)
    dv = dinv_ref[...]
    t2a = (agg_ref[0] + hs_ref[0]) * dv
    t2b = (agg_ref[1] + hs_ref[1]) * dv
    W2 = W2_ref[...]
    h2 = jnp.maximum(
        _mm(t2a, W2[0:32]) + _mm(t2b, W2[32:64]) + b2_ref[...], 0.0)
    means = [jnp.sum(h2[b * 500:(b + 1) * 500], axis=0, keepdims=True)
             * (1.0 / 500.0) for b in range(4)]
    m = jnp.concatenate(means, axis=0)
    out_ref[pl.ds(i * 4, 4), :] = _mm(m, fcW_ref[...]) + fcb_ref[...]


def _final_kernel(agg2, hs2, dinv, W2, b2r, fc_W, fcb_r):
    return pl.pallas_call(
        _k4_body,
        grid=(25,),
        in_specs=[
            pl.BlockSpec((2, 2000, 32), lambda i: (0, i, 0)),
            pl.BlockSpec((2, 2000, 32), lambda i: (0, i, 0)),
            pl.BlockSpec((2000, 1), lambda i: (i, 0)),
            pl.BlockSpec((64, 64), lambda i: (0, 0)),
            pl.BlockSpec((1, 64), lambda i: (0, 0)),
            pl.BlockSpec((64, 18), lambda i: (0, 0)),
            pl.BlockSpec((1, 18), lambda i: (0, 0)),
        ],
        out_specs=pl.BlockSpec((100, 18), lambda i: (0, 0)),
        out_shape=jax.ShapeDtypeStruct((100, 18), jnp.float32),
    )(agg2, hs2, dinv, W2, b2r, fc_W, fcb_r)


def kernel(x, edge_index, conv1_w, conv1_b, conv2_w, conv2_b,
           W1, b1, W2, b2, fc_W, fc_b):
    f32 = jnp.float32
    xq = jnp.pad(x, ((0, 0), (2, 2), (0, 0))).reshape(100, 501, 36)
    w1t = jnp.transpose(conv1_w, (2, 1, 0))
    z9 = jnp.zeros((9, 16), f32)
    wA = jnp.stack([jnp.concatenate(
        [w1t[o - p] if 0 <= o - p < 5 else z9 for o in range(4)], axis=0)
        for p in range(4)])
    wB = jnp.stack([jnp.concatenate(
        [w1t[o - p] if 0 <= o - p < 5 else z9 for o in range(4, 8)], axis=0)
        for p in range(4)])
    w2s = jnp.transpose(conv2_w, (2, 1, 0))
    b1r = conv1_b.reshape(1, 16)
    b2r = conv2_b.reshape(1, 32)

    src = edge_index[0].astype(jnp.int32)
    dst = edge_index[1].astype(jnp.int32)
    pad_ids = N + (jnp.arange(EPAD - E, dtype=jnp.int32) % (NPAD - N))
    srcp = jnp.concatenate([src, pad_ids]).reshape(ECH, 128)
    dstp = jnp.concatenate([dst, pad_ids]).reshape(ECH, 128)
    srcp2 = jnp.stack([srcp, srcp + NPAD])

    gb1 = b1.reshape(1, 64)
    gb2 = b2.reshape(1, 64)
    fcb = fc_b.reshape(1, 18)

    h0 = _conv_stack(xq, wA, wB, w2s, b1r, b2r)
    h0p = jnp.pad(h0.reshape(N, 32), ((0, NPAD - N), (0, 0)))
    deg2 = _sc_degree(dstp).reshape(2, NPAD)
    deg2t = jnp.transpose(deg2)
    dinv, hs1 = _scale_kernel(deg2t, h0p)
    agg1 = _sc_agg(hs1, srcp, dstp, feat_split=False)
    hs2 = _h1_kernel(agg1, hs1, dinv, W1, gb1)
    hs2cat = hs2.reshape(2 * NPAD, 32)
    agg2 = _sc_agg(hs2cat, srcp2, dstp, feat_split=True)
    return _final_kernel(agg2, hs2, dinv, W2, gb2, fc_W, fcb)


# speedup vs baseline: 15.7108x; 1.1854x over previous
import functools

import jax
import jax.numpy as jnp
from jax import lax
from jax.experimental import pallas as pl
from jax.experimental.pallas import tpu as pltpu
from jax.experimental.pallas import tpu_sc as plsc

N = 50000
NPAD = 51200
E = 800000
EPAD = 819200
ECH = EPAD // 128
CPW32 = ECH // 32
CPW16 = ECH // 16
RPS = NPAD // 16


def _mm(a, b):
    return jax.lax.dot_general(
        a, b, (((1,), (0,)), ((), ())),
        precision=jax.lax.Precision.HIGHEST,
        preferred_element_type=jnp.float32)


def _k1_body(xq_ref, wA_ref, wB_ref, w2s_ref, b1_ref, b2_ref, out_ref,
             p1e_ref, p1o_ref):
    xq = xq_ref[0]
    x0 = xq[0:500]
    x1 = xq[1:501]
    b1v = b1_ref[...]
    c = [jnp.maximum(_mm(x0, wA_ref[p]) + _mm(x1, wB_ref[p]) + b1v, 0.0)
         for p in range(4)]
    p1e_ref[0:1, :] = jnp.zeros((1, 16), jnp.float32)
    p1e_ref[1:501, :] = jnp.maximum(c[0], c[1])
    p1e_ref[501:502, :] = jnp.zeros((1, 16), jnp.float32)
    p1o_ref[0:1, :] = jnp.zeros((1, 16), jnp.float32)
    p1o_ref[1:501, :] = jnp.maximum(c[2], c[3])
    p1o_ref[501:502, :] = jnp.zeros((1, 16), jnp.float32)
    e_m1 = p1e_ref[0:500]
    e_0 = p1e_ref[1:501]
    e_p1 = p1e_ref[2:502]
    o_m1 = p1o_ref[0:500]
    o_0 = p1o_ref[1:501]
    o_p1 = p1o_ref[2:502]
    b2v = b2_ref[...]
    c2e = (_mm(e_m1, w2s_ref[0]) + _mm(o_m1, w2s_ref[1]) + _mm(e_0, w2s_ref[2])
           + _mm(o_0, w2s_ref[3]) + _mm(e_p1, w2s_ref[4]) + b2v)
    c2o = (_mm(o_m1, w2s_ref[0]) + _mm(e_0, w2s_ref[1]) + _mm(o_0, w2s_ref[2])
           + _mm(e_p1, w2s_ref[3]) + _mm(o_p1, w2s_ref[4]) + b2v)
    out_ref[0] = jnp.maximum(jnp.maximum(c2e, 0.0), jnp.maximum(c2o, 0.0))


def _conv_stack(xq, wA, wB, w2s, b1r, b2r):
    return pl.pallas_call(
        _k1_body,
        grid=(100,),
        in_specs=[
            pl.BlockSpec((1, 501, 36), lambda i: (i, 0, 0)),
            pl.BlockSpec((4, 36, 16), lambda i: (0, 0, 0)),
            pl.BlockSpec((4, 36, 16), lambda i: (0, 0, 0)),
            pl.BlockSpec((5, 16, 32), lambda i: (0, 0, 0)),
            pl.BlockSpec((1, 16), lambda i: (0, 0)),
            pl.BlockSpec((1, 32), lambda i: (0, 0)),
        ],
        out_specs=pl.BlockSpec((1, 500, 32), lambda i: (i, 0, 0)),
        out_shape=jax.ShapeDtypeStruct((100, 500, 32), jnp.float32),
        scratch_shapes=[pltpu.VMEM((502, 16), jnp.float32),
                        pltpu.VMEM((502, 16), jnp.float32)],
    )(xq, wA, wB, w2s, b1r, b2r)


def _sc_degree(dstp):
    mesh = plsc.VectorSubcoreMesh(core_axis_name="c", subcore_axis_name="s")

    @functools.partial(
        pl.kernel,
        out_type=jax.ShapeDtypeStruct((2, 1, NPAD), jnp.float32),
        mesh=mesh,
        compiler_params=pltpu.CompilerParams(use_tc_tiling_on_sc=False),
        scratch_types=[
            pltpu.VMEM((CPW32, 128), jnp.int32),
            pltpu.VMEM((128,), jnp.float32),
            pltpu.VMEM((RPS,), jnp.float32),
            pltpu.VMEM_SHARED((NPAD,), jnp.float32),
        ])
    def deg_k(dst_hbm, out_hbm, idx_v, ones_v, zs_v, acc_sh):
        cid = lax.axis_index("c")
        sid = lax.axis_index("s")
        w = cid * 16 + sid

        def fill_ones(i, _):
            ones_v[pl.ds(i * 16, 16)] = jnp.ones((16,), jnp.float32)
            return 0
        lax.fori_loop(0, 8, fill_ones, 0)

        def fill_z(i, _):
            zs_v[pl.ds(i * 16, 16)] = jnp.zeros((16,), jnp.float32)
            return 0
        lax.fori_loop(0, RPS // 16, fill_z, 0)

        pltpu.sync_copy(zs_v, acc_sh.at[pl.ds(sid * RPS, RPS)])
        plsc.subcore_barrier()
        pltpu.sync_copy(dst_hbm.at[pl.ds(w * CPW32, CPW32)], idx_v)

        def body(j, _):
            pltpu.sync_copy(ones_v, acc_sh.at[idx_v.at[j]], add=True)
            return 0
        lax.fori_loop(0, CPW32, body, 0)

        plsc.subcore_barrier()
        pltpu.sync_copy(acc_sh.at[pl.ds(sid * RPS, RPS)],
                        out_hbm.at[cid, 0, pl.ds(sid * RPS, RPS)])

    return deg_k(dstp)


def _sc_agg(tab, srcp, dstp, feat_split):
    mesh = plsc.VectorSubcoreMesh(core_axis_name="c", subcore_axis_name="s")
    cpw = CPW16 if feat_split else CPW32
    TCH = 50
    NBUF = 2
    ntiles = cpw // TCH

    @functools.partial(
        pl.kernel,
        out_type=jax.ShapeDtypeStruct((2, NPAD, 32), jnp.float32),
        mesh=mesh,
        compiler_params=pltpu.CompilerParams(use_tc_tiling_on_sc=False),
        scratch_types=[
            pltpu.VMEM((TCH, 128), jnp.int32),
            pltpu.VMEM((TCH, 128), jnp.int32),
            pltpu.VMEM((NBUF, 128, 32), jnp.float32),
            pltpu.VMEM((100, 32), jnp.float32),
            pltpu.VMEM_SHARED((NPAD, 32), jnp.float32),
        ] + [pltpu.SemaphoreType.DMA] * NBUF)
    def agg_k(tab_hbm, src_hbm, dst_hbm, out_hbm, srcv, dstv, rows_v, zr_v,
              acc_sh, *sems):
        cid = lax.axis_index("c")
        sid = lax.axis_index("s")

        def fill_z(i, _):
            zr_v[i, pl.ds(0, 16)] = jnp.zeros((16,), jnp.float32)
            zr_v[i, pl.ds(16, 16)] = jnp.zeros((16,), jnp.float32)
            return 0
        lax.fori_loop(0, 100, fill_z, 0)

        def zero_acc(k, _):
            pltpu.sync_copy(zr_v, acc_sh.at[pl.ds(sid * RPS + k * 100, 100)])
            return 0
        lax.fori_loop(0, 32, zero_acc, 0)
        plsc.subcore_barrier()

        def tile_body(t, _):
            if feat_split:
                base = sid * cpw + t * TCH
                pltpu.sync_copy(src_hbm.at[cid, pl.ds(base, TCH)], srcv)
                pltpu.sync_copy(dst_hbm.at[pl.ds(base, TCH)], dstv)
            else:
                base = (cid * 16 + sid) * cpw + t * TCH
                pltpu.sync_copy(src_hbm.at[pl.ds(base, TCH)], srcv)
                pltpu.sync_copy(dst_hbm.at[pl.ds(base, TCH)], dstv)

            for b in range(NBUF):
                pltpu.async_copy(tab_hbm.at[srcv.at[b]], rows_v.at[b],
                                 sems[b])

            def body(g, _):
                j0 = g * NBUF
                for b in range(NBUF):
                    pltpu.make_async_copy(
                        tab_hbm.at[srcv.at[j0 + b]], rows_v.at[b],
                        sems[b]).wait()
                    pltpu.sync_copy(rows_v.at[b],
                                    acc_sh.at[dstv.at[j0 + b]], add=True)
                    pltpu.async_copy(
                        tab_hbm.at[srcv.at[j0 + b + NBUF]], rows_v.at[b],
                        sems[b])
                return 0
            lax.fori_loop(0, TCH // NBUF - 1, body, 0)

            j0 = TCH - NBUF
            for b in range(NBUF):
                pltpu.make_async_copy(
                    tab_hbm.at[srcv.at[j0 + b]], rows_v.at[b], sems[b]).wait()
                pltpu.sync_copy(rows_v.at[b],
                                acc_sh.at[dstv.at[j0 + b]], add=True)
            return 0
        lax.fori_loop(0, ntiles, tile_body, 0)

        plsc.subcore_barrier()
        pltpu.sync_copy(acc_sh.at[pl.ds(sid * RPS, RPS)],
                        out_hbm.at[cid, pl.ds(sid * RPS, RPS)])

    return agg_k(tab, srcp, dstp)


def _k2_body(degt_ref, h0_ref, dinv_ref, hs1_ref):
    d = degt_ref[:, 0:1] + degt_ref[:, 1:2] + 1.0
    dv = lax.rsqrt(d)
    dinv_ref[...] = dv
    hs1_ref[...] = h0_ref[...] * dv


def _scale_kernel(deg2t, h0p):
    return pl.pallas_call(
        _k2_body,
        grid=(64,),
        in_specs=[
            pl.BlockSpec((800, 2), lambda i: (i, 0)),
            pl.BlockSpec((800, 32), lambda i: (i, 0)),
        ],
        out_specs=[
            pl.BlockSpec((800, 1), lambda i: (i, 0)),
            pl.BlockSpec((800, 32), lambda i: (i, 0)),
        ],
        out_shape=[jax.ShapeDtypeStruct((NPAD, 1), jnp.float32),
                   jax.ShapeDtypeStruct((NPAD, 32), jnp.float32)],
    )(deg2t, h0p)


def _k3_body(agg_ref, hs1_ref, dinv_ref, W1_ref, b1_ref, out_ref):
    i = pl.program_id(0)
    dv = dinv_ref[...]
    t1 = (agg_ref[0] + agg_ref[1] + hs1_ref[...]) * dv
    h1 = jnp.maximum(_mm(t1, W1_ref[...]) + b1_ref[...], 0.0)
    rows = i * 800 + lax.broadcasted_iota(jnp.int32, (800, 1), 0)
    hs2 = jnp.where(rows < N, h1 * dv, 0.0)
    out_ref[0] = hs2[:, 0:32]
    out_ref[1] = hs2[:, 32:64]


def _h1_kernel(agg1, hs1, dinv, W1, b1r):
    return pl.pallas_call(
        _k3_body,
        grid=(64,),
        in_specs=[
            pl.BlockSpec((2, 800, 32), lambda i: (0, i, 0)),
            pl.BlockSpec((800, 32), lambda i: (i, 0)),
            pl.BlockSpec((800, 1), lambda i: (i, 0)),
            pl.BlockSpec((32, 64), lambda i: (0, 0)),
            pl.BlockSpec((1, 64), lambda i: (0, 0)),
        ],
        out_specs=pl.BlockSpec((2, 800, 32), lambda i: (0, i, 0)),
        out_shape=jax.ShapeDtypeStruct((2, NPAD, 32), jnp.float32),
    )(agg1, hs1, dinv, W1, b1r)


def _k4_body(agg_ref, hs_ref, dinv_ref, W2_ref, b2_ref, fcW_ref, fcb_ref,
             out_ref):
    i = pl.program_id(0)
    dv = dinv_ref[...]
    t2a = (agg_ref[0] + hs_ref[0]) * dv
    t2b = (agg_ref[1] + hs_ref[1]) * dv
    W2 = W2_ref[...]
    h2 = jnp.maximum(
        _mm(t2a, W2[0:32]) + _mm(t2b, W2[32:64]) + b2_ref[...], 0.0)
    means = [jnp.sum(h2[b * 500:(b + 1) * 500], axis=0, keepdims=True)
             * (1.0 / 500.0) for b in range(4)]
    m = jnp.concatenate(means, axis=0)
    out_ref[pl.ds(i * 4, 4), :] = _mm(m, fcW_ref[...]) + fcb_ref[...]


def _final_kernel(agg2, hs2, dinv, W2, b2r, fc_W, fcb_r):
    return pl.pallas_call(
        _k4_body,
        grid=(25,),
        in_specs=[
            pl.BlockSpec((2, 2000, 32), lambda i: (0, i, 0)),
            pl.BlockSpec((2, 2000, 32), lambda i: (0, i, 0)),
            pl.BlockSpec((2000, 1), lambda i: (i, 0)),
            pl.BlockSpec((64, 64), lambda i: (0, 0)),
            pl.BlockSpec((1, 64), lambda i: (0, 0)),
            pl.BlockSpec((64, 18), lambda i: (0, 0)),
            pl.BlockSpec((1, 18), lambda i: (0, 0)),
        ],
        out_specs=pl.BlockSpec((100, 18), lambda i: (0, 0)),
        out_shape=jax.ShapeDtypeStruct((100, 18), jnp.float32),
    )(agg2, hs2, dinv, W2, b2r, fc_W, fcb_r)


def kernel(x, edge_index, conv1_w, conv1_b, conv2_w, conv2_b,
           W1, b1, W2, b2, fc_W, fc_b):
    f32 = jnp.float32
    xq = jnp.pad(x, ((0, 0), (2, 2), (0, 0))).reshape(100, 501, 36)
    w1t = jnp.transpose(conv1_w, (2, 1, 0))
    z9 = jnp.zeros((9, 16), f32)
    wA = jnp.stack([jnp.concatenate(
        [w1t[o - p] if 0 <= o - p < 5 else z9 for o in range(4)], axis=0)
        for p in range(4)])
    wB = jnp.stack([jnp.concatenate(
        [w1t[o - p] if 0 <= o - p < 5 else z9 for o in range(4, 8)], axis=0)
        for p in range(4)])
    w2s = jnp.transpose(conv2_w, (2, 1, 0))
    b1r = conv1_b.reshape(1, 16)
    b2r = conv2_b.reshape(1, 32)

    src = edge_index[0].astype(jnp.int32)
    dst = edge_index[1].astype(jnp.int32)
    pad_ids = N + (jnp.arange(EPAD - E, dtype=jnp.int32) % (NPAD - N))
    srcp = jnp.concatenate([src, pad_ids]).reshape(ECH, 128)
    dstp = jnp.concatenate([dst, pad_ids]).reshape(ECH, 128)
    srcp2 = jnp.stack([srcp, srcp + NPAD])

    gb1 = b1.reshape(1, 64)
    gb2 = b2.reshape(1, 64)
    fcb = fc_b.reshape(1, 18)

    h0 = _conv_stack(xq, wA, wB, w2s, b1r, b2r)
    h0p = jnp.pad(h0.reshape(N, 32), ((0, NPAD - N), (0, 0)))
    deg2 = _sc_degree(dstp).reshape(2, NPAD)
    deg2t = jnp.transpose(deg2)
    dinv, hs1 = _scale_kernel(deg2t, h0p)
    agg1 = _sc_agg(hs1, srcp, dstp, feat_split=False)
    hs2 = _h1_kernel(agg1, hs1, dinv, W1, gb1)
    hs2cat = hs2.reshape(2 * NPAD, 32)
    agg2 = _sc_agg(hs2cat, srcp2, dstp, feat_split=True)
    return _final_kernel(agg2, hs2, dinv, W2, gb2, fc_W, fcb)


# speedup vs baseline: 16.9940x; 1.0817x over previous
import functools

import jax
import jax.numpy as jnp
from jax import lax
from jax.experimental import pallas as pl
from jax.experimental.pallas import tpu as pltpu
from jax.experimental.pallas import tpu_sc as plsc

N = 50000
NPAD = 51200
E = 800000
EPAD = 819200
ECH = EPAD // 128
CPW32 = ECH // 32
CPW16 = ECH // 16
RPS = NPAD // 16


def _mm(a, b):
    return jax.lax.dot_general(
        a, b, (((1,), (0,)), ((), ())),
        precision=jax.lax.Precision.HIGHEST,
        preferred_element_type=jnp.float32)


def _k1_body(xq_ref, wA_ref, wB_ref, w2s_ref, b1_ref, b2_ref, out_ref,
             p1e_ref, p1o_ref):
    xq = xq_ref[0]
    x0 = xq[0:500]
    x1 = xq[1:501]
    b1v = b1_ref[...]
    c = [jnp.maximum(_mm(x0, wA_ref[p]) + _mm(x1, wB_ref[p]) + b1v, 0.0)
         for p in range(4)]
    p1e_ref[0:1, :] = jnp.zeros((1, 16), jnp.float32)
    p1e_ref[1:501, :] = jnp.maximum(c[0], c[1])
    p1e_ref[501:502, :] = jnp.zeros((1, 16), jnp.float32)
    p1o_ref[0:1, :] = jnp.zeros((1, 16), jnp.float32)
    p1o_ref[1:501, :] = jnp.maximum(c[2], c[3])
    p1o_ref[501:502, :] = jnp.zeros((1, 16), jnp.float32)
    e_m1 = p1e_ref[0:500]
    e_0 = p1e_ref[1:501]
    e_p1 = p1e_ref[2:502]
    o_m1 = p1o_ref[0:500]
    o_0 = p1o_ref[1:501]
    o_p1 = p1o_ref[2:502]
    b2v = b2_ref[...]
    c2e = (_mm(e_m1, w2s_ref[0]) + _mm(o_m1, w2s_ref[1]) + _mm(e_0, w2s_ref[2])
           + _mm(o_0, w2s_ref[3]) + _mm(e_p1, w2s_ref[4]) + b2v)
    c2o = (_mm(o_m1, w2s_ref[0]) + _mm(e_0, w2s_ref[1]) + _mm(o_0, w2s_ref[2])
           + _mm(e_p1, w2s_ref[3]) + _mm(o_p1, w2s_ref[4]) + b2v)
    out_ref[0] = jnp.maximum(jnp.maximum(c2e, 0.0), jnp.maximum(c2o, 0.0))


def _conv_stack(xq, wA, wB, w2s, b1r, b2r):
    return pl.pallas_call(
        _k1_body,
        grid=(100,),
        in_specs=[
            pl.BlockSpec((1, 501, 36), lambda i: (i, 0, 0)),
            pl.BlockSpec((4, 36, 16), lambda i: (0, 0, 0)),
            pl.BlockSpec((4, 36, 16), lambda i: (0, 0, 0)),
            pl.BlockSpec((5, 16, 32), lambda i: (0, 0, 0)),
            pl.BlockSpec((1, 16), lambda i: (0, 0)),
            pl.BlockSpec((1, 32), lambda i: (0, 0)),
        ],
        out_specs=pl.BlockSpec((1, 500, 32), lambda i: (i, 0, 0)),
        out_shape=jax.ShapeDtypeStruct((100, 500, 32), jnp.float32),
        scratch_shapes=[pltpu.VMEM((502, 16), jnp.float32),
                        pltpu.VMEM((502, 16), jnp.float32)],
    )(xq, wA, wB, w2s, b1r, b2r)


def _sc_degree(dstp):
    mesh = plsc.VectorSubcoreMesh(core_axis_name="c", subcore_axis_name="s")

    @functools.partial(
        pl.kernel,
        out_type=jax.ShapeDtypeStruct((2, 1, NPAD), jnp.float32),
        mesh=mesh,
        compiler_params=pltpu.CompilerParams(use_tc_tiling_on_sc=False),
        scratch_types=[
            pltpu.VMEM((CPW32, 128), jnp.int32),
            pltpu.VMEM((128,), jnp.float32),
            pltpu.VMEM((RPS,), jnp.float32),
            pltpu.VMEM_SHARED((NPAD,), jnp.float32),
        ])
    def deg_k(dst_hbm, out_hbm, idx_v, ones_v, zs_v, acc_sh):
        cid = lax.axis_index("c")
        sid = lax.axis_index("s")
        w = cid * 16 + sid

        def fill_ones(i, _):
            ones_v[pl.ds(i * 16, 16)] = jnp.ones((16,), jnp.float32)
            return 0
        lax.fori_loop(0, 8, fill_ones, 0)

        def fill_z(i, _):
            zs_v[pl.ds(i * 16, 16)] = jnp.zeros((16,), jnp.float32)
            return 0
        lax.fori_loop(0, RPS // 16, fill_z, 0)

        pltpu.sync_copy(zs_v, acc_sh.at[pl.ds(sid * RPS, RPS)])
        plsc.subcore_barrier()
        pltpu.sync_copy(dst_hbm.at[pl.ds(w * CPW32, CPW32)], idx_v)

        def body(j, _):
            pltpu.sync_copy(ones_v, acc_sh.at[idx_v.at[j]], add=True)
            return 0
        lax.fori_loop(0, CPW32, body, 0)

        plsc.subcore_barrier()
        pltpu.sync_copy(acc_sh.at[pl.ds(sid * RPS, RPS)],
                        out_hbm.at[cid, 0, pl.ds(sid * RPS, RPS)])

    return deg_k(dstp)


def _sc_agg(tab, srcp, dstp, feat_split):
    mesh = plsc.VectorSubcoreMesh(core_axis_name="c", subcore_axis_name="s")
    cpw = CPW16 if feat_split else CPW32
    TCH = 40
    NBUF = 4
    ntiles = cpw // TCH

    @functools.partial(
        pl.kernel,
        out_type=jax.ShapeDtypeStruct((2, NPAD, 32), jnp.float32),
        mesh=mesh,
        compiler_params=pltpu.CompilerParams(use_tc_tiling_on_sc=False),
        scratch_types=[
            pltpu.VMEM((TCH, 128), jnp.int32),
            pltpu.VMEM((TCH, 128), jnp.int32),
            pltpu.VMEM((NBUF, 128, 32), jnp.float32),
            pltpu.VMEM((50, 32), jnp.float32),
            pltpu.VMEM_SHARED((NPAD, 32), jnp.float32),
        ] + [pltpu.SemaphoreType.DMA] * NBUF)
    def agg_k(tab_hbm, src_hbm, dst_hbm, out_hbm, srcv, dstv, rows_v, zr_v,
              acc_sh, *sems):
        cid = lax.axis_index("c")
        sid = lax.axis_index("s")

        def fill_z(i, _):
            zr_v[i, pl.ds(0, 16)] = jnp.zeros((16,), jnp.float32)
            zr_v[i, pl.ds(16, 16)] = jnp.zeros((16,), jnp.float32)
            return 0
        lax.fori_loop(0, 50, fill_z, 0)

        def zero_acc(k, _):
            pltpu.sync_copy(zr_v, acc_sh.at[pl.ds(sid * RPS + k * 50, 50)])
            return 0
        lax.fori_loop(0, 64, zero_acc, 0)
        plsc.subcore_barrier()

        def tile_body(t, _):
            if feat_split:
                base = sid * cpw + t * TCH
                pltpu.sync_copy(src_hbm.at[cid, pl.ds(base, TCH)], srcv)
                pltpu.sync_copy(dst_hbm.at[pl.ds(base, TCH)], dstv)
            else:
                base = (cid * 16 + sid) * cpw + t * TCH
                pltpu.sync_copy(src_hbm.at[pl.ds(base, TCH)], srcv)
                pltpu.sync_copy(dst_hbm.at[pl.ds(base, TCH)], dstv)

            for b in range(NBUF):
                pltpu.async_copy(tab_hbm.at[srcv.at[b]], rows_v.at[b],
                                 sems[b])

            def body(g, _):
                j0 = g * NBUF
                for b in range(NBUF):
                    pltpu.make_async_copy(
                        tab_hbm.at[srcv.at[j0 + b]], rows_v.at[b],
                        sems[b]).wait()
                    pltpu.sync_copy(rows_v.at[b],
                                    acc_sh.at[dstv.at[j0 + b]], add=True)
                    pltpu.async_copy(
                        tab_hbm.at[srcv.at[j0 + b + NBUF]], rows_v.at[b],
                        sems[b])
                return 0
            lax.fori_loop(0, TCH // NBUF - 1, body, 0)

            j0 = TCH - NBUF
            for b in range(NBUF):
                pltpu.make_async_copy(
                    tab_hbm.at[srcv.at[j0 + b]], rows_v.at[b], sems[b]).wait()
                pltpu.sync_copy(rows_v.at[b],
                                acc_sh.at[dstv.at[j0 + b]], add=True)
            return 0
        lax.fori_loop(0, ntiles, tile_body, 0)

        plsc.subcore_barrier()
        pltpu.sync_copy(acc_sh.at[pl.ds(sid * RPS, RPS)],
                        out_hbm.at[cid, pl.ds(sid * RPS, RPS)])

    return agg_k(tab, srcp, dstp)


def _k2_body(degt_ref, h0_ref, dinv_ref, hs1_ref):
    d = degt_ref[:, 0:1] + degt_ref[:, 1:2] + 1.0
    dv = lax.rsqrt(d)
    dinv_ref[...] = dv
    hs1_ref[...] = h0_ref[...] * dv


def _scale_kernel(deg2t, h0p):
    return pl.pallas_call(
        _k2_body,
        grid=(64,),
        in_specs=[
            pl.BlockSpec((800, 2), lambda i: (i, 0)),
            pl.BlockSpec((800, 32), lambda i: (i, 0)),
        ],
        out_specs=[
            pl.BlockSpec((800, 1), lambda i: (i, 0)),
            pl.BlockSpec((800, 32), lambda i: (i, 0)),
        ],
        out_shape=[jax.ShapeDtypeStruct((NPAD, 1), jnp.float32),
                   jax.ShapeDtypeStruct((NPAD, 32), jnp.float32)],
    )(deg2t, h0p)


def _k3_body(agg_ref, hs1_ref, dinv_ref, W1_ref, b1_ref, out_ref):
    i = pl.program_id(0)
    dv = dinv_ref[...]
    t1 = (agg_ref[0] + agg_ref[1] + hs1_ref[...]) * dv
    h1 = jnp.maximum(_mm(t1, W1_ref[...]) + b1_ref[...], 0.0)
    rows = i * 800 + lax.broadcasted_iota(jnp.int32, (800, 1), 0)
    hs2 = jnp.where(rows < N, h1 * dv, 0.0)
    out_ref[0] = hs2[:, 0:32]
    out_ref[1] = hs2[:, 32:64]


def _h1_kernel(agg1, hs1, dinv, W1, b1r):
    return pl.pallas_call(
        _k3_body,
        grid=(64,),
        in_specs=[
            pl.BlockSpec((2, 800, 32), lambda i: (0, i, 0)),
            pl.BlockSpec((800, 32), lambda i: (i, 0)),
            pl.BlockSpec((800, 1), lambda i: (i, 0)),
            pl.BlockSpec((32, 64), lambda i: (0, 0)),
            pl.BlockSpec((1, 64), lambda i: (0, 0)),
        ],
        out_specs=pl.BlockSpec((2, 800, 32), lambda i: (0, i, 0)),
        out_shape=jax.ShapeDtypeStruct((2, NPAD, 32), jnp.float32),
    )(agg1, hs1, dinv, W1, b1r)


def _k4_body(agg_ref, hs_ref, dinv_ref, W2_ref, b2_ref, fcW_ref, fcb_ref,
             out_ref):
    i = pl.program_id(0)
    dv = dinv_ref[...]
    t2a = (agg_ref[0] + hs_ref[0]) * dv
    t2b = (agg_ref[1] + hs_ref[1]) * dv
    W2 = W2_ref[...]
    h2 = jnp.maximum(
        _mm(t2a, W2[0:32]) + _mm(t2b, W2[32:64]) + b2_ref[...], 0.0)
    means = [jnp.sum(h2[b * 500:(b + 1) * 500], axis=0, keepdims=True)
             * (1.0 / 500.0) for b in range(4)]
    m = jnp.concatenate(means, axis=0)
    out_ref[pl.ds(i * 4, 4), :] = _mm(m, fcW_ref[...]) + fcb_ref[...]


def _final_kernel(agg2, hs2, dinv, W2, b2r, fc_W, fcb_r):
    return pl.pallas_call(
        _k4_body,
        grid=(25,),
        in_specs=[
            pl.BlockSpec((2, 2000, 32), lambda i: (0, i, 0)),
            pl.BlockSpec((2, 2000, 32), lambda i: (0, i, 0)),
            pl.BlockSpec((2000, 1), lambda i: (i, 0)),
            pl.BlockSpec((64, 64), lambda i: (0, 0)),
            pl.BlockSpec((1, 64), lambda i: (0, 0)),
            pl.BlockSpec((64, 18), lambda i: (0, 0)),
            pl.BlockSpec((1, 18), lambda i: (0, 0)),
        ],
        out_specs=pl.BlockSpec((100, 18), lambda i: (0, 0)),
        out_shape=jax.ShapeDtypeStruct((100, 18), jnp.float32),
    )(agg2, hs2, dinv, W2, b2r, fc_W, fcb_r)


def kernel(x, edge_index, conv1_w, conv1_b, conv2_w, conv2_b,
           W1, b1, W2, b2, fc_W, fc_b):
    f32 = jnp.float32
    xq = jnp.pad(x, ((0, 0), (2, 2), (0, 0))).reshape(100, 501, 36)
    w1t = jnp.transpose(conv1_w, (2, 1, 0))
    z9 = jnp.zeros((9, 16), f32)
    wA = jnp.stack([jnp.concatenate(
        [w1t[o - p] if 0 <= o - p < 5 else z9 for o in range(4)], axis=0)
        for p in range(4)])
    wB = jnp.stack([jnp.concatenate(
        [w1t[o - p] if 0 <= o - p < 5 else z9 for o in range(4, 8)], axis=0)
        for p in range(4)])
    w2s = jnp.transpose(conv2_w, (2, 1, 0))
    b1r = conv1_b.reshape(1, 16)
    b2r = conv2_b.reshape(1, 32)

    src = edge_index[0].astype(jnp.int32)
    dst = edge_index[1].astype(jnp.int32)
    pad_ids = N + (jnp.arange(EPAD - E, dtype=jnp.int32) % (NPAD - N))
    srcp = jnp.concatenate([src, pad_ids]).reshape(ECH, 128)
    dstp = jnp.concatenate([dst, pad_ids]).reshape(ECH, 128)
    srcp2 = jnp.stack([srcp, srcp + NPAD])

    gb1 = b1.reshape(1, 64)
    gb2 = b2.reshape(1, 64)
    fcb = fc_b.reshape(1, 18)

    h0 = _conv_stack(xq, wA, wB, w2s, b1r, b2r)
    h0p = jnp.pad(h0.reshape(N, 32), ((0, NPAD - N), (0, 0)))
    deg2 = _sc_degree(dstp).reshape(2, NPAD)
    deg2t = jnp.transpose(deg2)
    dinv, hs1 = _scale_kernel(deg2t, h0p)
    agg1 = _sc_agg(hs1, srcp, dstp, feat_split=False)
    hs2 = _h1_kernel(agg1, hs1, dinv, W1, gb1)
    hs2cat = hs2.reshape(2 * NPAD, 32)
    agg2 = _sc_agg(hs2cat, srcp2, dstp, feat_split=True)
    return _final_kernel(agg2, hs2, dinv, W2, gb2, fc_W, fcb)


# speedup vs baseline: 26.1669x; 1.5398x over previous
import functools

import jax
import jax.numpy as jnp
from jax import lax
from jax.experimental import pallas as pl
from jax.experimental.pallas import tpu as pltpu
from jax.experimental.pallas import tpu_sc as plsc

N = 50000
S = 512
NPAD = 51200
E = 800000
EPAD = 819200
ECH = EPAD // 128
CPW32 = ECH // 32
CPW16 = ECH // 16
RPS = NPAD // 16


def _mm(a, b):
    return jax.lax.dot_general(
        a, b, (((1,), (0,)), ((), ())),
        precision=jax.lax.Precision.HIGHEST,
        preferred_element_type=jnp.float32)


BR = 5120


def _k1_body(xq_ref, wA_ref, wB_ref, w2s_ref, b1_ref, b2_ref, out_ref,
             xs_ref, pe_ref, po_ref):
    xq = xq_ref[...]
    xs_ref[0:BR - 1] = xq[1:BR]
    xs_ref[BR - 1:BR] = jnp.zeros((1, 36), jnp.float32)
    cc = jnp.maximum(
        _mm(xq, wA_ref[...]) + _mm(xs_ref[...], wB_ref[...]) + b1_ref[...],
        0.0)
    w = lax.rem(lax.broadcasted_iota(jnp.int32, (BR, 1), 0), S)
    live = w < 500
    e = jnp.where(live, jnp.maximum(cc[:, 0:16], cc[:, 16:32]), 0.0)
    o = jnp.where(live, jnp.maximum(cc[:, 32:48], cc[:, 48:64]), 0.0)
    pe_ref[0:1] = jnp.zeros((1, 16), jnp.float32)
    pe_ref[1:BR + 1] = e
    pe_ref[BR + 1:BR + 2] = jnp.zeros((1, 16), jnp.float32)
    po_ref[0:1] = jnp.zeros((1, 16), jnp.float32)
    po_ref[1:BR + 1] = o
    po_ref[BR + 1:BR + 2] = jnp.zeros((1, 16), jnp.float32)
    e_m1 = pe_ref[0:BR]
    e_0 = pe_ref[1:BR + 1]
    e_p1 = pe_ref[2:BR + 2]
    o_m1 = po_ref[0:BR]
    o_0 = po_ref[1:BR + 1]
    o_p1 = po_ref[2:BR + 2]
    b2v = b2_ref[...]
    c2e = (_mm(e_m1, w2s_ref[0]) + _mm(o_m1, w2s_ref[1]) + _mm(e_0, w2s_ref[2])
           + _mm(o_0, w2s_ref[3]) + _mm(e_p1, w2s_ref[4]) + b2v)
    c2o = (_mm(o_m1, w2s_ref[0]) + _mm(e_0, w2s_ref[1]) + _mm(o_0, w2s_ref[2])
           + _mm(e_p1, w2s_ref[3]) + _mm(o_p1, w2s_ref[4]) + b2v)
    h0 = jnp.maximum(jnp.maximum(c2e, 0.0), jnp.maximum(c2o, 0.0))
    out_ref[...] = jnp.where(live, h0, 0.0)


def _conv_stack(xq, wA, wB, w2s, b1t, b2r):
    return pl.pallas_call(
        _k1_body,
        grid=(NPAD // BR,),
        in_specs=[
            pl.BlockSpec((BR, 36), lambda i: (i, 0)),
            pl.BlockSpec((36, 64), lambda i: (0, 0)),
            pl.BlockSpec((36, 64), lambda i: (0, 0)),
            pl.BlockSpec((5, 16, 32), lambda i: (0, 0, 0)),
            pl.BlockSpec((1, 64), lambda i: (0, 0)),
            pl.BlockSpec((1, 32), lambda i: (0, 0)),
        ],
        out_specs=pl.BlockSpec((BR, 32), lambda i: (i, 0)),
        out_shape=jax.ShapeDtypeStruct((NPAD, 32), jnp.float32),
        scratch_shapes=[pltpu.VMEM((BR, 36), jnp.float32),
                        pltpu.VMEM((BR + 2, 16), jnp.float32),
                        pltpu.VMEM((BR + 2, 16), jnp.float32)],
    )(xq, wA, wB, w2s, b1t, b2r)


def _sc_degree(dstp):
    mesh = plsc.VectorSubcoreMesh(core_axis_name="c", subcore_axis_name="s")

    @functools.partial(
        pl.kernel,
        out_type=jax.ShapeDtypeStruct((2, 1, NPAD), jnp.float32),
        mesh=mesh,
        compiler_params=pltpu.CompilerParams(use_tc_tiling_on_sc=False),
        scratch_types=[
            pltpu.VMEM((CPW32, 128), jnp.int32),
            pltpu.VMEM((128,), jnp.float32),
            pltpu.VMEM((RPS,), jnp.float32),
            pltpu.VMEM_SHARED((NPAD,), jnp.float32),
        ])
    def deg_k(dst_hbm, out_hbm, idx_v, ones_v, zs_v, acc_sh):
        cid = lax.axis_index("c")
        sid = lax.axis_index("s")
        w = cid * 16 + sid

        def fill_ones(i, _):
            ones_v[pl.ds(i * 16, 16)] = jnp.ones((16,), jnp.float32)
            return 0
        lax.fori_loop(0, 8, fill_ones, 0)

        def fill_z(i, _):
            zs_v[pl.ds(i * 16, 16)] = jnp.zeros((16,), jnp.float32)
            return 0
        lax.fori_loop(0, RPS // 16, fill_z, 0)

        pltpu.sync_copy(zs_v, acc_sh.at[pl.ds(sid * RPS, RPS)])
        plsc.subcore_barrier()
        pltpu.sync_copy(dst_hbm.at[pl.ds(w * CPW32, CPW32)], idx_v)

        def body(j, _):
            pltpu.sync_copy(ones_v, acc_sh.at[idx_v.at[j]], add=True)
            return 0
        lax.fori_loop(0, CPW32, body, 0)

        plsc.subcore_barrier()
        pltpu.sync_copy(acc_sh.at[pl.ds(sid * RPS, RPS)],
                        out_hbm.at[cid, 0, pl.ds(sid * RPS, RPS)])

    return deg_k(dstp)


def _sc_agg(tab, srcp, dstp, feat_split):
    mesh = plsc.VectorSubcoreMesh(core_axis_name="c", subcore_axis_name="s")
    cpw = CPW16 if feat_split else CPW32
    TCH = 40
    NBUF = 4
    ntiles = cpw // TCH

    @functools.partial(
        pl.kernel,
        out_type=jax.ShapeDtypeStruct((2, NPAD, 32), jnp.float32),
        mesh=mesh,
        compiler_params=pltpu.CompilerParams(use_tc_tiling_on_sc=False),
        scratch_types=[
            pltpu.VMEM((TCH, 128), jnp.int32),
            pltpu.VMEM((TCH, 128), jnp.int32),
            pltpu.VMEM((NBUF, 128, 32), jnp.float32),
            pltpu.VMEM((50, 32), jnp.float32),
            pltpu.VMEM_SHARED((NPAD, 32), jnp.float32),
        ] + [pltpu.SemaphoreType.DMA] * NBUF)
    def agg_k(tab_hbm, src_hbm, dst_hbm, out_hbm, srcv, dstv, rows_v, zr_v,
              acc_sh, *sems):
        cid = lax.axis_index("c")
        sid = lax.axis_index("s")

        def fill_z(i, _):
            zr_v[i, pl.ds(0, 16)] = jnp.zeros((16,), jnp.float32)
            zr_v[i, pl.ds(16, 16)] = jnp.zeros((16,), jnp.float32)
            return 0
        lax.fori_loop(0, 50, fill_z, 0)

        def zero_acc(k, _):
            pltpu.sync_copy(zr_v, acc_sh.at[pl.ds(sid * RPS + k * 50, 50)])
            return 0
        lax.fori_loop(0, 64, zero_acc, 0)
        plsc.subcore_barrier()

        def gsrc(j):
            return tab_hbm.at[srcv.at[j]]

        def tile_body(t, _):
            if feat_split:
                base = sid * cpw + t * TCH
                pltpu.sync_copy(src_hbm.at[cid, pl.ds(base, TCH)], srcv)
            else:
                base = (cid * 16 + sid) * cpw + t * TCH
                pltpu.sync_copy(src_hbm.at[pl.ds(base, TCH)], srcv)
            pltpu.sync_copy(dst_hbm.at[pl.ds(base, TCH)], dstv)

            for b in range(NBUF):
                pltpu.async_copy(gsrc(b), rows_v.at[b], sems[b])

            def body(g, _):
                j0 = g * NBUF
                for b in range(NBUF):
                    pltpu.make_async_copy(
                        gsrc(j0 + b), rows_v.at[b], sems[b]).wait()
                    pltpu.sync_copy(rows_v.at[b],
                                    acc_sh.at[dstv.at[j0 + b]], add=True)
                    pltpu.async_copy(
                        gsrc(j0 + b + NBUF), rows_v.at[b], sems[b])
                return 0
            lax.fori_loop(0, TCH // NBUF - 1, body, 0)

            j0 = TCH - NBUF
            for b in range(NBUF):
                pltpu.make_async_copy(
                    gsrc(j0 + b), rows_v.at[b], sems[b]).wait()
                pltpu.sync_copy(rows_v.at[b],
                                acc_sh.at[dstv.at[j0 + b]], add=True)
            return 0
        lax.fori_loop(0, ntiles, tile_body, 0)

        plsc.subcore_barrier()
        pltpu.sync_copy(acc_sh.at[pl.ds(sid * RPS, RPS)],
                        out_hbm.at[cid, pl.ds(sid * RPS, RPS)])

    return agg_k(tab, srcp, dstp)


def _k2_body(degt_ref, h0_ref, dinv_ref, hs1_ref):
    d = degt_ref[:, 0:1] + degt_ref[:, 1:2] + 1.0
    dv = lax.rsqrt(d)
    dinv_ref[...] = dv
    hs1_ref[...] = h0_ref[...] * dv


def _scale_kernel(deg2t, h0p):
    return pl.pallas_call(
        _k2_body,
        grid=(8,),
        in_specs=[
            pl.BlockSpec((6400, 2), lambda i: (i, 0)),
            pl.BlockSpec((6400, 32), lambda i: (i, 0)),
        ],
        out_specs=[
            pl.BlockSpec((6400, 1), lambda i: (i, 0)),
            pl.BlockSpec((6400, 32), lambda i: (i, 0)),
        ],
        out_shape=[jax.ShapeDtypeStruct((NPAD, 1), jnp.float32),
                   jax.ShapeDtypeStruct((NPAD, 32), jnp.float32)],
    )(deg2t, h0p)


def _k3_body(agg_ref, hs1_ref, dinv_ref, W1_ref, b1_ref, out_ref):
    i = pl.program_id(0)
    dv = dinv_ref[...]
    t1 = (agg_ref[0] + agg_ref[1] + hs1_ref[...]) * dv
    h1 = jnp.maximum(_mm(t1, W1_ref[...]) + b1_ref[...], 0.0)
    rows = i * 6400 + lax.broadcasted_iota(jnp.int32, (6400, 1), 0)
    live = lax.rem(rows, S) < 500
    hs2 = jnp.where(live, h1 * dv, 0.0)
    out_ref[0] = hs2[:, 0:32]
    out_ref[1] = hs2[:, 32:64]


def _h1_kernel(agg1, hs1, dinv, W1, b1r):
    return pl.pallas_call(
        _k3_body,
        grid=(8,),
        in_specs=[
            pl.BlockSpec((2, 6400, 32), lambda i: (0, i, 0)),
            pl.BlockSpec((6400, 32), lambda i: (i, 0)),
            pl.BlockSpec((6400, 1), lambda i: (i, 0)),
            pl.BlockSpec((32, 64), lambda i: (0, 0)),
            pl.BlockSpec((1, 64), lambda i: (0, 0)),
        ],
        out_specs=pl.BlockSpec((2, 6400, 32), lambda i: (0, i, 0)),
        out_shape=jax.ShapeDtypeStruct((2, NPAD, 32), jnp.float32),
    )(agg1, hs1, dinv, W1, b1r)


def _k4_body(agg_ref, hs_ref, dinv_ref, W2_ref, b2_ref, fcW_ref, fcb_ref,
             out_ref):
    i = pl.program_id(0)
    dv = dinv_ref[...]
    t2a = (agg_ref[0] + hs_ref[0]) * dv
    t2b = (agg_ref[1] + hs_ref[1]) * dv
    W2 = W2_ref[...]
    h2 = jnp.maximum(
        _mm(t2a, W2[0:32]) + _mm(t2b, W2[32:64]) + b2_ref[...], 0.0)
    means = [jnp.sum(h2[b * S:b * S + 500], axis=0, keepdims=True)
             * (1.0 / 500.0) for b in range(4)]
    m = jnp.concatenate(means, axis=0)
    out_ref[pl.ds(i * 4, 4), :] = _mm(m, fcW_ref[...]) + fcb_ref[...]


def _final_kernel(agg2, hs2, dinv, W2, b2r, fc_W, fcb_r):
    return pl.pallas_call(
        _k4_body,
        grid=(25,),
        in_specs=[
            pl.BlockSpec((2, 2048, 32), lambda i: (0, i, 0)),
            pl.BlockSpec((2, 2048, 32), lambda i: (0, i, 0)),
            pl.BlockSpec((2048, 1), lambda i: (i, 0)),
            pl.BlockSpec((64, 64), lambda i: (0, 0)),
            pl.BlockSpec((1, 64), lambda i: (0, 0)),
            pl.BlockSpec((64, 18), lambda i: (0, 0)),
            pl.BlockSpec((1, 18), lambda i: (0, 0)),
        ],
        out_specs=pl.BlockSpec((100, 18), lambda i: (0, 0)),
        out_shape=jax.ShapeDtypeStruct((100, 18), jnp.float32),
    )(agg2, hs2, dinv, W2, b2r, fc_W, fcb_r)


def kernel(x, edge_index, conv1_w, conv1_b, conv2_w, conv2_b,
           W1, b1, W2, b2, fc_W, fc_b):
    f32 = jnp.float32
    xg = jnp.pad(x, ((0, 0), (2, 2), (0, 0))).reshape(100, 501, 36)
    xq = jnp.pad(xg, ((0, 0), (0, S - 501), (0, 0))).reshape(NPAD, 36)
    w1t = jnp.transpose(conv1_w, (2, 1, 0))
    z9 = jnp.zeros((9, 16), f32)
    wA = jnp.concatenate([jnp.concatenate(
        [w1t[o - p] if 0 <= o - p < 5 else z9 for o in range(4)], axis=0)
        for p in range(4)], axis=1)
    wB = jnp.concatenate([jnp.concatenate(
        [w1t[o - p] if 0 <= o - p < 5 else z9 for o in range(4, 8)], axis=0)
        for p in range(4)], axis=1)
    w2s = jnp.transpose(conv2_w, (2, 1, 0))
    b1t = jnp.tile(conv1_b.reshape(1, 16), (1, 4))
    b2r = conv2_b.reshape(1, 32)

    src = edge_index[0].astype(jnp.int32)
    dst = edge_index[1].astype(jnp.int32)
    src = src + 12 * (src // 500)
    dst = dst + 12 * (dst // 500)
    ji = jnp.arange(EPAD - E, dtype=jnp.int32)
    pad_ids = S * (ji % 100) + 500 + ((ji // 100) % 12)
    srcp = jnp.concatenate([src, pad_ids]).reshape(ECH, 128)
    dstp = jnp.concatenate([dst, pad_ids]).reshape(ECH, 128)
    srcp2 = jnp.stack([srcp, srcp + NPAD])

    gb1 = b1.reshape(1, 64)
    gb2 = b2.reshape(1, 64)
    fcb = fc_b.reshape(1, 18)

    h0 = _conv_stack(xq, wA, wB, w2s, b1t, b2r)
    deg2 = _sc_degree(dstp).reshape(2, NPAD)
    deg2t = jnp.transpose(deg2)
    dinv, hs1 = _scale_kernel(deg2t, h0)
    agg1 = _sc_agg(hs1, srcp, dstp, feat_split=False)
    hs2 = _h1_kernel(agg1, hs1, dinv, W1, gb1)
    agg2 = _sc_agg(hs2.reshape(2 * NPAD, 32), srcp2, dstp, feat_split=True)
    return _final_kernel(agg2, hs2, dinv, W2, gb2, fc_W, fcb)


# speedup vs baseline: 29.4830x; 1.1267x over previous
import functools

import jax
import jax.numpy as jnp
from jax import lax
from jax.experimental import pallas as pl
from jax.experimental.pallas import tpu as pltpu
from jax.experimental.pallas import tpu_sc as plsc

N = 50000
S = 512
NPAD = 51200
E = 800000
EPAD = 819200
ECH = EPAD // 128
CPW32 = ECH // 32
CPW16 = ECH // 16
RPS = NPAD // 16


def _mm(a, b):
    return jax.lax.dot_general(
        a, b, (((1,), (0,)), ((), ())),
        precision=jax.lax.Precision.HIGHEST,
        preferred_element_type=jnp.float32)


BR = 5120


def _k1_body(xq_ref, wA_ref, wB_ref, wm_ref, w0_ref, wp_ref, b1_ref,
             b2_ref, out_ref, xs_ref, peo_ref):
    xq = xq_ref[...]
    xs_ref[0:BR - 1] = xq[1:BR]
    xs_ref[BR - 1:BR] = jnp.zeros((1, 36), jnp.float32)
    cc = jnp.maximum(
        _mm(xq, wA_ref[...]) + _mm(xs_ref[...], wB_ref[...]) + b1_ref[...],
        0.0)
    w = lax.rem(lax.broadcasted_iota(jnp.int32, (BR, 1), 0), S)
    live = w < 500
    eo = jnp.where(live, jnp.maximum(cc[:, 0:32], cc[:, 32:64]), 0.0)
    peo_ref[0:1] = jnp.zeros((1, 32), jnp.float32)
    peo_ref[1:BR + 1] = eo
    peo_ref[BR + 1:BR + 2] = jnp.zeros((1, 32), jnp.float32)
    c2 = (_mm(peo_ref[0:BR], wm_ref[...]) + _mm(peo_ref[1:BR + 1], w0_ref[...])
          + _mm(peo_ref[2:BR + 2], wp_ref[...]) + b2_ref[...])
    h0 = jnp.maximum(jnp.maximum(c2[:, 0:32], c2[:, 32:64]), 0.0)
    out_ref[...] = jnp.where(live, h0, 0.0)


def _conv_stack(xq, wA, wB, wm, w0, wp, b1t, b2t):
    return pl.pallas_call(
        _k1_body,
        grid=(NPAD // BR,),
        in_specs=[
            pl.BlockSpec((BR, 36), lambda i: (i, 0)),
            pl.BlockSpec((36, 64), lambda i: (0, 0)),
            pl.BlockSpec((36, 64), lambda i: (0, 0)),
            pl.BlockSpec((32, 64), lambda i: (0, 0)),
            pl.BlockSpec((32, 64), lambda i: (0, 0)),
            pl.BlockSpec((32, 64), lambda i: (0, 0)),
            pl.BlockSpec((1, 64), lambda i: (0, 0)),
            pl.BlockSpec((1, 64), lambda i: (0, 0)),
        ],
        out_specs=pl.BlockSpec((BR, 32), lambda i: (i, 0)),
        out_shape=jax.ShapeDtypeStruct((NPAD, 32), jnp.float32),
        scratch_shapes=[pltpu.VMEM((BR, 36), jnp.float32),
                        pltpu.VMEM((BR + 2, 32), jnp.float32)],
    )(xq, wA, wB, wm, w0, wp, b1t, b2t)


def _sc_degree(dstp):
    mesh = plsc.VectorSubcoreMesh(core_axis_name="c", subcore_axis_name="s")

    @functools.partial(
        pl.kernel,
        out_type=jax.ShapeDtypeStruct((2, 1, NPAD), jnp.float32),
        mesh=mesh,
        compiler_params=pltpu.CompilerParams(use_tc_tiling_on_sc=False),
        scratch_types=[
            pltpu.VMEM((CPW32, 128), jnp.int32),
            pltpu.VMEM((128,), jnp.float32),
            pltpu.VMEM((RPS,), jnp.float32),
            pltpu.VMEM_SHARED((NPAD,), jnp.float32),
        ])
    def deg_k(dst_hbm, out_hbm, idx_v, ones_v, zs_v, acc_sh):
        cid = lax.axis_index("c")
        sid = lax.axis_index("s")
        w = cid * 16 + sid

        def fill_ones(i, _):
            ones_v[pl.ds(i * 16, 16)] = jnp.ones((16,), jnp.float32)
            return 0
        lax.fori_loop(0, 8, fill_ones, 0)

        def fill_z(i, _):
            zs_v[pl.ds(i * 16, 16)] = jnp.zeros((16,), jnp.float32)
            return 0
        lax.fori_loop(0, RPS // 16, fill_z, 0)

        pltpu.sync_copy(zs_v, acc_sh.at[pl.ds(sid * RPS, RPS)])
        plsc.subcore_barrier()
        pltpu.sync_copy(dst_hbm.at[pl.ds(w * CPW32, CPW32)], idx_v)

        def body(j, _):
            pltpu.sync_copy(ones_v, acc_sh.at[idx_v.at[j]], add=True)
            return 0
        lax.fori_loop(0, CPW32, body, 0)

        plsc.subcore_barrier()
        pltpu.sync_copy(acc_sh.at[pl.ds(sid * RPS, RPS)],
                        out_hbm.at[cid, 0, pl.ds(sid * RPS, RPS)])

    return deg_k(dstp)


def _sc_agg(tab, srcp, dstp, feat_split):
    mesh = plsc.VectorSubcoreMesh(core_axis_name="c", subcore_axis_name="s")
    cpw = CPW16 if feat_split else CPW32
    TCH = 40
    NBUF = 4
    ntiles = cpw // TCH

    @functools.partial(
        pl.kernel,
        out_type=jax.ShapeDtypeStruct((2, NPAD, 32), jnp.float32),
        mesh=mesh,
        compiler_params=pltpu.CompilerParams(use_tc_tiling_on_sc=False),
        scratch_types=[
            pltpu.VMEM((TCH, 128), jnp.int32),
            pltpu.VMEM((TCH, 128), jnp.int32),
            pltpu.VMEM((NBUF, 128, 32), jnp.float32),
            pltpu.VMEM((50, 32), jnp.float32),
            pltpu.VMEM_SHARED((NPAD, 32), jnp.float32),
        ] + [pltpu.SemaphoreType.DMA] * NBUF)
    def agg_k(tab_hbm, src_hbm, dst_hbm, out_hbm, srcv, dstv, rows_v, zr_v,
              acc_sh, *sems):
        cid = lax.axis_index("c")
        sid = lax.axis_index("s")

        def fill_z(i, _):
            zr_v[i, pl.ds(0, 16)] = jnp.zeros((16,), jnp.float32)
            zr_v[i, pl.ds(16, 16)] = jnp.zeros((16,), jnp.float32)
            return 0
        lax.fori_loop(0, 50, fill_z, 0)

        def zero_acc(k, _):
            pltpu.sync_copy(zr_v, acc_sh.at[pl.ds(sid * RPS + k * 50, 50)])
            return 0
        lax.fori_loop(0, 64, zero_acc, 0)
        plsc.subcore_barrier()

        def gsrc(j):
            return tab_hbm.at[srcv.at[j]]

        def tile_body(t, _):
            if feat_split:
                base = sid * cpw + t * TCH
                pltpu.sync_copy(src_hbm.at[cid, pl.ds(base, TCH)], srcv)
            else:
                base = (cid * 16 + sid) * cpw + t * TCH
                pltpu.sync_copy(src_hbm.at[pl.ds(base, TCH)], srcv)
            pltpu.sync_copy(dst_hbm.at[pl.ds(base, TCH)], dstv)

            for b in range(NBUF):
                pltpu.async_copy(gsrc(b), rows_v.at[b], sems[b])

            def body(g, _):
                j0 = g * NBUF
                for b in range(NBUF):
                    pltpu.make_async_copy(
                        gsrc(j0 + b), rows_v.at[b], sems[b]).wait()
                    pltpu.sync_copy(rows_v.at[b],
                                    acc_sh.at[dstv.at[j0 + b]], add=True)
                    pltpu.async_copy(
                        gsrc(j0 + b + NBUF), rows_v.at[b], sems[b])
                return 0
            lax.fori_loop(0, TCH // NBUF - 1, body, 0)

            j0 = TCH - NBUF
            for b in range(NBUF):
                pltpu.make_async_copy(
                    gsrc(j0 + b), rows_v.at[b], sems[b]).wait()
                pltpu.sync_copy(rows_v.at[b],
                                acc_sh.at[dstv.at[j0 + b]], add=True)
            return 0
        lax.fori_loop(0, ntiles, tile_body, 0)

        plsc.subcore_barrier()
        pltpu.sync_copy(acc_sh.at[pl.ds(sid * RPS, RPS)],
                        out_hbm.at[cid, pl.ds(sid * RPS, RPS)])

    return agg_k(tab, srcp, dstp)


def _k2_body(degt_ref, h0_ref, dinv_ref, hs1_ref):
    d = degt_ref[:, 0:1] + degt_ref[:, 1:2] + 1.0
    dv = lax.rsqrt(d)
    dinv_ref[...] = dv
    hs1_ref[...] = h0_ref[...] * dv


def _scale_kernel(deg2t, h0p):
    return pl.pallas_call(
        _k2_body,
        grid=(8,),
        in_specs=[
            pl.BlockSpec((6400, 2), lambda i: (i, 0)),
            pl.BlockSpec((6400, 32), lambda i: (i, 0)),
        ],
        out_specs=[
            pl.BlockSpec((6400, 1), lambda i: (i, 0)),
            pl.BlockSpec((6400, 32), lambda i: (i, 0)),
        ],
        out_shape=[jax.ShapeDtypeStruct((NPAD, 1), jnp.float32),
                   jax.ShapeDtypeStruct((NPAD, 32), jnp.float32)],
    )(deg2t, h0p)


def _k3_body(agg_ref, hs1_ref, dinv_ref, W1_ref, b1_ref, out_ref):
    i = pl.program_id(0)
    dv = dinv_ref[...]
    t1 = (agg_ref[0] + agg_ref[1] + hs1_ref[...]) * dv
    h1 = jnp.maximum(_mm(t1, W1_ref[...]) + b1_ref[...], 0.0)
    rows = i * 6400 + lax.broadcasted_iota(jnp.int32, (6400, 1), 0)
    live = lax.rem(rows, S) < 500
    hs2 = jnp.where(live, h1 * dv, 0.0)
    out_ref[0] = hs2[:, 0:32]
    out_ref[1] = hs2[:, 32:64]


def _h1_kernel(agg1, hs1, dinv, W1, b1r):
    return pl.pallas_call(
        _k3_body,
        grid=(8,),
        in_specs=[
            pl.BlockSpec((2, 6400, 32), lambda i: (0, i, 0)),
            pl.BlockSpec((6400, 32), lambda i: (i, 0)),
            pl.BlockSpec((6400, 1), lambda i: (i, 0)),
            pl.BlockSpec((32, 64), lambda i: (0, 0)),
            pl.BlockSpec((1, 64), lambda i: (0, 0)),
        ],
        out_specs=pl.BlockSpec((2, 6400, 32), lambda i: (0, i, 0)),
        out_shape=jax.ShapeDtypeStruct((2, NPAD, 32), jnp.float32),
    )(agg1, hs1, dinv, W1, b1r)


def _k4_body(agg_ref, hs_ref, dinv_ref, W2_ref, b2_ref, fcW_ref, fcb_ref,
             out_ref):
    i = pl.program_id(0)
    dv = dinv_ref[...]
    t2a = (agg_ref[0] + hs_ref[0]) * dv
    t2b = (agg_ref[1] + hs_ref[1]) * dv
    W2 = W2_ref[...]
    h2 = jnp.maximum(
        _mm(t2a, W2[0:32]) + _mm(t2b, W2[32:64]) + b2_ref[...], 0.0)
    means = [jnp.sum(h2[b * S:b * S + 500], axis=0, keepdims=True)
             * (1.0 / 500.0) for b in range(4)]
    m = jnp.concatenate(means, axis=0)
    out_ref[pl.ds(i * 4, 4), :] = _mm(m, fcW_ref[...]) + fcb_ref[...]


def _final_kernel(agg2, hs2, dinv, W2, b2r, fc_W, fcb_r):
    return pl.pallas_call(
        _k4_body,
        grid=(25,),
        in_specs=[
            pl.BlockSpec((2, 2048, 32), lambda i: (0, i, 0)),
            pl.BlockSpec((2, 2048, 32), lambda i: (0, i, 0)),
            pl.BlockSpec((2048, 1), lambda i: (i, 0)),
            pl.BlockSpec((64, 64), lambda i: (0, 0)),
            pl.BlockSpec((1, 64), lambda i: (0, 0)),
            pl.BlockSpec((64, 18), lambda i: (0, 0)),
            pl.BlockSpec((1, 18), lambda i: (0, 0)),
        ],
        out_specs=pl.BlockSpec((100, 18), lambda i: (0, 0)),
        out_shape=jax.ShapeDtypeStruct((100, 18), jnp.float32),
    )(agg2, hs2, dinv, W2, b2r, fc_W, fcb_r)


def kernel(x, edge_index, conv1_w, conv1_b, conv2_w, conv2_b,
           W1, b1, W2, b2, fc_W, fc_b):
    f32 = jnp.float32
    xg = jnp.pad(x, ((0, 0), (2, 2), (0, 0))).reshape(100, 501, 36)
    xq = jnp.pad(xg, ((0, 0), (0, S - 501), (0, 0))).reshape(NPAD, 36)
    w1t = jnp.transpose(conv1_w, (2, 1, 0))
    z9 = jnp.zeros((9, 16), f32)
    wA = jnp.concatenate([jnp.concatenate(
        [w1t[o - p] if 0 <= o - p < 5 else z9 for o in range(4)], axis=0)
        for p in (0, 2, 1, 3)], axis=1)
    wB = jnp.concatenate([jnp.concatenate(
        [w1t[o - p] if 0 <= o - p < 5 else z9 for o in range(4, 8)], axis=0)
        for p in (0, 2, 1, 3)], axis=1)
    w2s = jnp.transpose(conv2_w, (2, 1, 0))
    z16 = jnp.zeros((16, 32), f32)
    cat = jnp.concatenate
    wm = cat([cat([w2s[0], z16], 1), cat([w2s[1], w2s[0]], 1)], 0)
    w0 = cat([cat([w2s[2], w2s[1]], 1), cat([w2s[3], w2s[2]], 1)], 0)
    wp = cat([cat([w2s[4], w2s[3]], 1), cat([z16, w2s[4]], 1)], 0)
    b1t = jnp.tile(conv1_b.reshape(1, 16), (1, 4))
    b2t = jnp.tile(conv2_b.reshape(1, 32), (1, 2))

    src = edge_index[0].astype(jnp.int32)
    dst = edge_index[1].astype(jnp.int32)
    src = src + 12 * (src // 500)
    dst = dst + 12 * (dst // 500)
    ji = jnp.arange(EPAD - E, dtype=jnp.int32)
    pad_ids = S * (ji % 100) + 500 + ((ji // 100) % 12)
    srcp = jnp.concatenate([src, pad_ids]).reshape(ECH, 128)
    dstp = jnp.concatenate([dst, pad_ids]).reshape(ECH, 128)
    srcp2 = jnp.stack([srcp, srcp + NPAD])

    gb1 = b1.reshape(1, 64)
    gb2 = b2.reshape(1, 64)
    fcb = fc_b.reshape(1, 18)

    h0 = _conv_stack(xq, wA, wB, wm, w0, wp, b1t, b2t)
    deg2 = _sc_degree(dstp).reshape(2, NPAD)
    deg2t = jnp.transpose(deg2)
    dinv, hs1 = _scale_kernel(deg2t, h0)
    agg1 = _sc_agg(hs1, srcp, dstp, feat_split=False)
    hs2 = _h1_kernel(agg1, hs1, dinv, W1, gb1)
    agg2 = _sc_agg(hs2.reshape(2 * NPAD, 32), srcp2, dstp, feat_split=True)
    return _final_kernel(agg2, hs2, dinv, W2, gb2, fc_W, fcb)


# speedup vs baseline: 36.4967x; 1.2379x over previous
import functools

import jax
import jax.numpy as jnp
from jax import lax
from jax.experimental import pallas as pl
from jax.experimental.pallas import tpu as pltpu
from jax.experimental.pallas import tpu_sc as plsc

N = 50000
S = 512
NPAD = 51200
E = 800000
EPAD = 819200
ECH = EPAD // 128
CPW32 = ECH // 32
CPW16 = ECH // 16
RPS = NPAD // 16


def _mm(a, b, prec=jax.lax.Precision.HIGHEST):
    return jax.lax.dot_general(
        a, b, (((1,), (0,)), ((), ())),
        precision=prec,
        preferred_element_type=jnp.float32)


def _mmd(a, b):
    return _mm(a, b, jax.lax.Precision.DEFAULT)


BR = 5120


def _k1_body(xq_ref, wA_ref, wB_ref, wm_ref, w0_ref, wp_ref, b1_ref,
             b2_ref, out_ref, xs_ref, peo_ref):
    xq = xq_ref[...]
    xs_ref[0:BR - 1] = xq[1:BR]
    xs_ref[BR - 1:BR] = jnp.zeros((1, 36), jnp.float32)
    cc = jnp.maximum(
        _mmd(xq, wA_ref[...]) + _mmd(xs_ref[...], wB_ref[...]) + b1_ref[...],
        0.0)
    w = lax.rem(lax.broadcasted_iota(jnp.int32, (BR, 1), 0), S)
    live = w < 500
    eo = jnp.where(live, jnp.maximum(cc[:, 0:32], cc[:, 32:64]), 0.0)
    peo_ref[0:1] = jnp.zeros((1, 32), jnp.float32)
    peo_ref[1:BR + 1] = eo
    peo_ref[BR + 1:BR + 2] = jnp.zeros((1, 32), jnp.float32)
    c2 = (_mmd(peo_ref[0:BR], wm_ref[...])
          + _mmd(peo_ref[1:BR + 1], w0_ref[...])
          + _mmd(peo_ref[2:BR + 2], wp_ref[...]) + b2_ref[...])
    h0 = jnp.maximum(jnp.maximum(c2[:, 0:32], c2[:, 32:64]), 0.0)
    out_ref[...] = jnp.where(live, h0, 0.0)


def _conv_stack(xq, wA, wB, wm, w0, wp, b1t, b2t):
    return pl.pallas_call(
        _k1_body,
        grid=(NPAD // BR,),
        in_specs=[
            pl.BlockSpec((BR, 36), lambda i: (i, 0)),
            pl.BlockSpec((36, 64), lambda i: (0, 0)),
            pl.BlockSpec((36, 64), lambda i: (0, 0)),
            pl.BlockSpec((32, 64), lambda i: (0, 0)),
            pl.BlockSpec((32, 64), lambda i: (0, 0)),
            pl.BlockSpec((32, 64), lambda i: (0, 0)),
            pl.BlockSpec((1, 64), lambda i: (0, 0)),
            pl.BlockSpec((1, 64), lambda i: (0, 0)),
        ],
        out_specs=pl.BlockSpec((BR, 32), lambda i: (i, 0)),
        out_shape=jax.ShapeDtypeStruct((NPAD, 32), jnp.float32),
        scratch_shapes=[pltpu.VMEM((BR, 36), jnp.float32),
                        pltpu.VMEM((BR + 2, 32), jnp.float32)],
    )(xq, wA, wB, wm, w0, wp, b1t, b2t)


def _sc_degree(dstp):
    mesh = plsc.VectorSubcoreMesh(core_axis_name="c", subcore_axis_name="s")

    @functools.partial(
        pl.kernel,
        out_type=jax.ShapeDtypeStruct((2, 1, NPAD), jnp.float32),
        mesh=mesh,
        compiler_params=pltpu.CompilerParams(use_tc_tiling_on_sc=False),
        scratch_types=[
            pltpu.VMEM((CPW32, 128), jnp.int32),
            pltpu.VMEM((128,), jnp.float32),
            pltpu.VMEM((RPS,), jnp.float32),
            pltpu.VMEM_SHARED((NPAD,), jnp.float32),
        ])
    def deg_k(dst_hbm, out_hbm, idx_v, ones_v, zs_v, acc_sh):
        cid = lax.axis_index("c")
        sid = lax.axis_index("s")
        w = cid * 16 + sid

        def fill_ones(i, _):
            ones_v[pl.ds(i * 16, 16)] = jnp.ones((16,), jnp.float32)
            return 0
        lax.fori_loop(0, 8, fill_ones, 0)

        def fill_z(i, _):
            zs_v[pl.ds(i * 16, 16)] = jnp.zeros((16,), jnp.float32)
            return 0
        lax.fori_loop(0, RPS // 16, fill_z, 0)

        pltpu.sync_copy(zs_v, acc_sh.at[pl.ds(sid * RPS, RPS)])
        plsc.subcore_barrier()
        pltpu.sync_copy(dst_hbm.at[pl.ds(w * CPW32, CPW32)], idx_v)

        def body(j, _):
            pltpu.sync_copy(ones_v, acc_sh.at[idx_v.at[j]], add=True)
            return 0
        lax.fori_loop(0, CPW32, body, 0)

        plsc.subcore_barrier()
        pltpu.sync_copy(acc_sh.at[pl.ds(sid * RPS, RPS)],
                        out_hbm.at[cid, 0, pl.ds(sid * RPS, RPS)])

    return deg_k(dstp)


def _sc_agg(tab, srcp, dstp, feat_split):
    mesh = plsc.VectorSubcoreMesh(core_axis_name="c", subcore_axis_name="s")
    cpw = CPW16 if feat_split else CPW32
    TCH = 40
    NBUF = 4
    ntiles = cpw // TCH

    @functools.partial(
        pl.kernel,
        out_type=jax.ShapeDtypeStruct((2, NPAD, 32), jnp.float32),
        mesh=mesh,
        compiler_params=pltpu.CompilerParams(use_tc_tiling_on_sc=False),
        scratch_types=[
            pltpu.VMEM((TCH, 128), jnp.int32),
            pltpu.VMEM((TCH, 128), jnp.int32),
            pltpu.VMEM((NBUF, 128, 32), jnp.float32),
            pltpu.VMEM((50, 32), jnp.float32),
            pltpu.VMEM_SHARED((NPAD, 32), jnp.float32),
        ] + [pltpu.SemaphoreType.DMA] * NBUF)
    def agg_k(tab_hbm, src_hbm, dst_hbm, out_hbm, srcv, dstv, rows_v, zr_v,
              acc_sh, *sems):
        cid = lax.axis_index("c")
        sid = lax.axis_index("s")

        def fill_z(i, _):
            zr_v[i, pl.ds(0, 16)] = jnp.zeros((16,), jnp.float32)
            zr_v[i, pl.ds(16, 16)] = jnp.zeros((16,), jnp.float32)
            return 0
        lax.fori_loop(0, 50, fill_z, 0)

        def zero_acc(k, _):
            pltpu.sync_copy(zr_v, acc_sh.at[pl.ds(sid * RPS + k * 50, 50)])
            return 0
        lax.fori_loop(0, 64, zero_acc, 0)
        plsc.subcore_barrier()

        def gsrc(j):
            return tab_hbm.at[srcv.at[j]]

        def tile_body(t, _):
            if feat_split:
                base = sid * cpw + t * TCH
                pltpu.sync_copy(src_hbm.at[cid, pl.ds(base, TCH)], srcv)
            else:
                base = (cid * 16 + sid) * cpw + t * TCH
                pltpu.sync_copy(src_hbm.at[pl.ds(base, TCH)], srcv)
            pltpu.sync_copy(dst_hbm.at[pl.ds(base, TCH)], dstv)

            for b in range(NBUF):
                pltpu.async_copy(gsrc(b), rows_v.at[b], sems[b])

            def body(g, _):
                j0 = g * NBUF
                for b in range(NBUF):
                    pltpu.make_async_copy(
                        gsrc(j0 + b), rows_v.at[b], sems[b]).wait()
                    pltpu.sync_copy(rows_v.at[b],
                                    acc_sh.at[dstv.at[j0 + b]], add=True)
                    pltpu.async_copy(
                        gsrc(j0 + b + NBUF), rows_v.at[b], sems[b])
                return 0
            lax.fori_loop(0, TCH // NBUF - 1, body, 0)

            j0 = TCH - NBUF
            for b in range(NBUF):
                pltpu.make_async_copy(
                    gsrc(j0 + b), rows_v.at[b], sems[b]).wait()
                pltpu.sync_copy(rows_v.at[b],
                                acc_sh.at[dstv.at[j0 + b]], add=True)
            return 0
        lax.fori_loop(0, ntiles, tile_body, 0)

        plsc.subcore_barrier()
        pltpu.sync_copy(acc_sh.at[pl.ds(sid * RPS, RPS)],
                        out_hbm.at[cid, pl.ds(sid * RPS, RPS)])

    return agg_k(tab, srcp, dstp)


def _k2_body(degt_ref, h0_ref, dinv_ref, hs1_ref):
    d = degt_ref[:, 0:1] + degt_ref[:, 1:2] + 1.0
    dv = lax.rsqrt(d)
    dinv_ref[...] = dv
    hs1_ref[...] = h0_ref[...] * dv


def _scale_kernel(deg2t, h0p):
    return pl.pallas_call(
        _k2_body,
        grid=(8,),
        in_specs=[
            pl.BlockSpec((6400, 2), lambda i: (i, 0)),
            pl.BlockSpec((6400, 32), lambda i: (i, 0)),
        ],
        out_specs=[
            pl.BlockSpec((6400, 1), lambda i: (i, 0)),
            pl.BlockSpec((6400, 32), lambda i: (i, 0)),
        ],
        out_shape=[jax.ShapeDtypeStruct((NPAD, 1), jnp.float32),
                   jax.ShapeDtypeStruct((NPAD, 32), jnp.float32)],
    )(deg2t, h0p)


def _k3_body(agg_ref, hs1_ref, dinv_ref, W1_ref, b1_ref, out_ref):
    i = pl.program_id(0)
    dv = dinv_ref[...]
    t1 = (agg_ref[0] + agg_ref[1] + hs1_ref[...]) * dv
    h1 = jnp.maximum(_mm(t1, W1_ref[...]) + b1_ref[...], 0.0)
    rows = i * 6400 + lax.broadcasted_iota(jnp.int32, (6400, 1), 0)
    live = lax.rem(rows, S) < 500
    hs2 = jnp.where(live, h1 * dv, 0.0)
    out_ref[0] = hs2[:, 0:32]
    out_ref[1] = hs2[:, 32:64]


def _h1_kernel(agg1, hs1, dinv, W1, b1r):
    return pl.pallas_call(
        _k3_body,
        grid=(8,),
        in_specs=[
            pl.BlockSpec((2, 6400, 32), lambda i: (0, i, 0)),
            pl.BlockSpec((6400, 32), lambda i: (i, 0)),
            pl.BlockSpec((6400, 1), lambda i: (i, 0)),
            pl.BlockSpec((32, 64), lambda i: (0, 0)),
            pl.BlockSpec((1, 64), lambda i: (0, 0)),
        ],
        out_specs=pl.BlockSpec((2, 6400, 32), lambda i: (0, i, 0)),
        out_shape=jax.ShapeDtypeStruct((2, NPAD, 32), jnp.float32),
    )(agg1, hs1, dinv, W1, b1r)


def _k4_body(agg_ref, hs_ref, dinv_ref, W2_ref, b2_ref, fcW_ref, fcb_ref,
             out_ref):
    i = pl.program_id(0)
    dv = dinv_ref[...]
    t2a = (agg_ref[0] + hs_ref[0]) * dv
    t2b = (agg_ref[1] + hs_ref[1]) * dv
    W2 = W2_ref[...]
    h2 = jnp.maximum(
        _mm(t2a, W2[0:32]) + _mm(t2b, W2[32:64]) + b2_ref[...], 0.0)
    means = [jnp.sum(h2[b * S:b * S + 500], axis=0, keepdims=True)
             * (1.0 / 500.0) for b in range(4)]
    m = jnp.concatenate(means, axis=0)
    out_ref[pl.ds(i * 4, 4), :] = _mm(m, fcW_ref[...]) + fcb_ref[...]


def _final_kernel(agg2, hs2, dinv, W2, b2r, fc_W, fcb_r):
    return pl.pallas_call(
        _k4_body,
        grid=(25,),
        in_specs=[
            pl.BlockSpec((2, 2048, 32), lambda i: (0, i, 0)),
            pl.BlockSpec((2, 2048, 32), lambda i: (0, i, 0)),
            pl.BlockSpec((2048, 1), lambda i: (i, 0)),
            pl.BlockSpec((64, 64), lambda i: (0, 0)),
            pl.BlockSpec((1, 64), lambda i: (0, 0)),
            pl.BlockSpec((64, 18), lambda i: (0, 0)),
            pl.BlockSpec((1, 18), lambda i: (0, 0)),
        ],
        out_specs=pl.BlockSpec((100, 18), lambda i: (0, 0)),
        out_shape=jax.ShapeDtypeStruct((100, 18), jnp.float32),
    )(agg2, hs2, dinv, W2, b2r, fc_W, fcb_r)


def kernel(x, edge_index, conv1_w, conv1_b, conv2_w, conv2_b,
           W1, b1, W2, b2, fc_W, fc_b):
    f32 = jnp.float32
    xg = jnp.pad(x, ((0, 0), (2, 2), (0, 0))).reshape(100, 501, 36)
    xq = jnp.pad(xg, ((0, 0), (0, S - 501), (0, 0))).reshape(NPAD, 36)
    w1t = jnp.transpose(conv1_w, (2, 1, 0))
    z9 = jnp.zeros((9, 16), f32)
    wA = jnp.concatenate([jnp.concatenate(
        [w1t[o - p] if 0 <= o - p < 5 else z9 for o in range(4)], axis=0)
        for p in (0, 2, 1, 3)], axis=1)
    wB = jnp.concatenate([jnp.concatenate(
        [w1t[o - p] if 0 <= o - p < 5 else z9 for o in range(4, 8)], axis=0)
        for p in (0, 2, 1, 3)], axis=1)
    w2s = jnp.transpose(conv2_w, (2, 1, 0))
    z16 = jnp.zeros((16, 32), f32)
    cat = jnp.concatenate
    wm = cat([cat([w2s[0], z16], 1), cat([w2s[1], w2s[0]], 1)], 0)
    w0 = cat([cat([w2s[2], w2s[1]], 1), cat([w2s[3], w2s[2]], 1)], 0)
    wp = cat([cat([w2s[4], w2s[3]], 1), cat([z16, w2s[4]], 1)], 0)
    b1t = jnp.tile(conv1_b.reshape(1, 16), (1, 4))
    b2t = jnp.tile(conv2_b.reshape(1, 32), (1, 2))

    src = edge_index[0].astype(jnp.int32)
    dst = edge_index[1].astype(jnp.int32)
    src = src + 12 * (src // 500)
    dst = dst + 12 * (dst // 500)
    ji = jnp.arange(EPAD - E, dtype=jnp.int32)
    pad_ids = S * (ji % 100) + 500 + ((ji // 100) % 12)
    srcp = jnp.concatenate([src, pad_ids]).reshape(ECH, 128)
    dstp = jnp.concatenate([dst, pad_ids]).reshape(ECH, 128)
    srcp2 = jnp.stack([srcp, srcp + NPAD])

    gb1 = b1.reshape(1, 64)
    gb2 = b2.reshape(1, 64)
    fcb = fc_b.reshape(1, 18)

    h0 = _conv_stack(xq, wA, wB, wm, w0, wp, b1t, b2t)
    deg2 = _sc_degree(dstp).reshape(2, NPAD)
    deg2t = jnp.transpose(deg2)
    dinv, hs1 = _scale_kernel(deg2t, h0)
    agg1 = _sc_agg(hs1, srcp, dstp, feat_split=False)
    hs2 = _h1_kernel(agg1, hs1, dinv, W1, gb1)
    agg2 = _sc_agg(hs2.reshape(2 * NPAD, 32), srcp2, dstp, feat_split=True)
    return _final_kernel(agg2, hs2, dinv, W2, gb2, fc_W, fcb)


# speedup vs baseline: 37.7498x; 1.0343x over previous
import functools

import jax
import jax.numpy as jnp
from jax import lax
from jax.experimental import pallas as pl
from jax.experimental.pallas import tpu as pltpu
from jax.experimental.pallas import tpu_sc as plsc

N = 50000
S = 512
NPAD = 51200
E = 800000
EPAD = 819200
ECH = EPAD // 128
CPW32 = ECH // 32
CPW16 = ECH // 16
RPS = NPAD // 16


def _mm(a, b, prec=jax.lax.Precision.HIGHEST):
    return jax.lax.dot_general(
        a, b, (((1,), (0,)), ((), ())),
        precision=prec,
        preferred_element_type=jnp.float32)


def _mmd(a, b):
    return _mm(a, b, jax.lax.Precision.DEFAULT)


BR = 5120


def _k1_body(xq_ref, wA_ref, wB_ref, wm_ref, w0_ref, wp_ref, b1_ref,
             b2_ref, out_ref, xs_ref, peo_ref):
    xq = xq_ref[...]
    xs_ref[0:BR - 1] = xq[1:BR]
    xs_ref[BR - 1:BR] = jnp.zeros((1, 36), jnp.float32)
    cc = jnp.maximum(
        _mmd(xq, wA_ref[...]) + _mmd(xs_ref[...], wB_ref[...]) + b1_ref[...],
        0.0)
    w = lax.rem(lax.broadcasted_iota(jnp.int32, (BR, 1), 0), S)
    live = w < 500
    eo = jnp.where(live, jnp.maximum(cc[:, 0:32], cc[:, 32:64]), 0.0)
    peo_ref[0:1] = jnp.zeros((1, 32), jnp.float32)
    peo_ref[1:BR + 1] = eo
    peo_ref[BR + 1:BR + 2] = jnp.zeros((1, 32), jnp.float32)
    c2 = (_mmd(peo_ref[0:BR], wm_ref[...])
          + _mmd(peo_ref[1:BR + 1], w0_ref[...])
          + _mmd(peo_ref[2:BR + 2], wp_ref[...]) + b2_ref[...])
    h0 = jnp.maximum(jnp.maximum(c2[:, 0:32], c2[:, 32:64]), 0.0)
    out_ref[...] = jnp.where(live, h0, 0.0)


def _conv_stack(xq, wA, wB, wm, w0, wp, b1t, b2t):
    return pl.pallas_call(
        _k1_body,
        grid=(NPAD // BR,),
        in_specs=[
            pl.BlockSpec((BR, 36), lambda i: (i, 0)),
            pl.BlockSpec((36, 64), lambda i: (0, 0)),
            pl.BlockSpec((36, 64), lambda i: (0, 0)),
            pl.BlockSpec((32, 64), lambda i: (0, 0)),
            pl.BlockSpec((32, 64), lambda i: (0, 0)),
            pl.BlockSpec((32, 64), lambda i: (0, 0)),
            pl.BlockSpec((1, 64), lambda i: (0, 0)),
            pl.BlockSpec((1, 64), lambda i: (0, 0)),
        ],
        out_specs=pl.BlockSpec((BR, 32), lambda i: (i, 0)),
        out_shape=jax.ShapeDtypeStruct((NPAD, 32), jnp.float32),
        scratch_shapes=[pltpu.VMEM((BR, 36), jnp.float32),
                        pltpu.VMEM((BR + 2, 32), jnp.float32)],
    )(xq, wA, wB, wm, w0, wp, b1t, b2t)


def _sc_degree(dstp):
    mesh = plsc.VectorSubcoreMesh(core_axis_name="c", subcore_axis_name="s")

    @functools.partial(
        pl.kernel,
        out_type=jax.ShapeDtypeStruct((2, 1, NPAD), jnp.float32),
        mesh=mesh,
        compiler_params=pltpu.CompilerParams(use_tc_tiling_on_sc=False),
        scratch_types=[
            pltpu.VMEM((CPW32, 128), jnp.int32),
            pltpu.VMEM((128,), jnp.float32),
            pltpu.VMEM((RPS,), jnp.float32),
            pltpu.VMEM_SHARED((NPAD,), jnp.float32),
        ])
    def deg_k(dst_hbm, out_hbm, idx_v, ones_v, zs_v, acc_sh):
        cid = lax.axis_index("c")
        sid = lax.axis_index("s")
        w = cid * 16 + sid

        def fill_ones(i, _):
            ones_v[pl.ds(i * 16, 16)] = jnp.ones((16,), jnp.float32)
            return 0
        lax.fori_loop(0, 8, fill_ones, 0)

        def fill_z(i, _):
            zs_v[pl.ds(i * 16, 16)] = jnp.zeros((16,), jnp.float32)
            return 0
        lax.fori_loop(0, RPS // 16, fill_z, 0)

        pltpu.sync_copy(zs_v, acc_sh.at[pl.ds(sid * RPS, RPS)])
        plsc.subcore_barrier()
        pltpu.sync_copy(dst_hbm.at[pl.ds(w * CPW32, CPW32)], idx_v)

        def body(j, _):
            pltpu.sync_copy(ones_v, acc_sh.at[idx_v.at[j]], add=True)
            return 0
        lax.fori_loop(0, CPW32, body, 0)

        plsc.subcore_barrier()
        pltpu.sync_copy(acc_sh.at[pl.ds(sid * RPS, RPS)],
                        out_hbm.at[cid, 0, pl.ds(sid * RPS, RPS)])

    return deg_k(dstp)


def _sc_agg(tab, srcp, dstp, feat_split):
    mesh = plsc.VectorSubcoreMesh(core_axis_name="c", subcore_axis_name="s")
    cpw = CPW16 if feat_split else CPW32
    TCH = 40
    NBUF = 4
    ntiles = cpw // TCH

    @functools.partial(
        pl.kernel,
        out_type=jax.ShapeDtypeStruct((2, NPAD, 32), jnp.float32),
        mesh=mesh,
        compiler_params=pltpu.CompilerParams(use_tc_tiling_on_sc=False),
        scratch_types=[
            pltpu.VMEM((TCH, 128), jnp.int32),
            pltpu.VMEM((TCH, 128), jnp.int32),
            pltpu.VMEM((NBUF, 128, 32), jnp.float32),
            pltpu.VMEM((50, 32), jnp.float32),
            pltpu.VMEM_SHARED((NPAD, 32), jnp.float32),
        ] + [pltpu.SemaphoreType.DMA] * NBUF)
    def agg_k(tab_hbm, src_hbm, dst_hbm, out_hbm, srcv, dstv, rows_v, zr_v,
              acc_sh, *sems):
        cid = lax.axis_index("c")
        sid = lax.axis_index("s")

        def fill_z(i, _):
            zr_v[i, pl.ds(0, 16)] = jnp.zeros((16,), jnp.float32)
            zr_v[i, pl.ds(16, 16)] = jnp.zeros((16,), jnp.float32)
            return 0
        lax.fori_loop(0, 50, fill_z, 0)

        def zero_acc(k, _):
            pltpu.sync_copy(zr_v, acc_sh.at[pl.ds(sid * RPS + k * 50, 50)])
            return 0
        lax.fori_loop(0, 64, zero_acc, 0)
        plsc.subcore_barrier()

        def gsrc(j):
            return tab_hbm.at[srcv.at[j]]

        def tile_body(t, _):
            if feat_split:
                base = sid * cpw + t * TCH
                pltpu.sync_copy(src_hbm.at[cid, pl.ds(base, TCH)], srcv)
            else:
                base = (cid * 16 + sid) * cpw + t * TCH
                pltpu.sync_copy(src_hbm.at[pl.ds(base, TCH)], srcv)
            pltpu.sync_copy(dst_hbm.at[pl.ds(base, TCH)], dstv)

            for b in range(NBUF):
                pltpu.async_copy(gsrc(b), rows_v.at[b], sems[b])

            def body(g, _):
                j0 = g * NBUF
                for b in range(NBUF):
                    pltpu.make_async_copy(
                        gsrc(j0 + b), rows_v.at[b], sems[b]).wait()
                    pltpu.sync_copy(rows_v.at[b],
                                    acc_sh.at[dstv.at[j0 + b]], add=True)
                    pltpu.async_copy(
                        gsrc(j0 + b + NBUF), rows_v.at[b], sems[b])
                return 0
            lax.fori_loop(0, TCH // NBUF - 1, body, 0)

            j0 = TCH - NBUF
            for b in range(NBUF):
                pltpu.make_async_copy(
                    gsrc(j0 + b), rows_v.at[b], sems[b]).wait()
                pltpu.sync_copy(rows_v.at[b],
                                acc_sh.at[dstv.at[j0 + b]], add=True)
            return 0
        lax.fori_loop(0, ntiles, tile_body, 0)

        plsc.subcore_barrier()
        pltpu.sync_copy(acc_sh.at[pl.ds(sid * RPS, RPS)],
                        out_hbm.at[cid, pl.ds(sid * RPS, RPS)])

    return agg_k(tab, srcp, dstp)


def _k2_body(degt_ref, h0_ref, dinv_ref, hs1_ref):
    d = degt_ref[:, 0:1] + degt_ref[:, 1:2] + 1.0
    dv = lax.rsqrt(d)
    dinv_ref[...] = dv
    hs1_ref[...] = h0_ref[...] * dv


def _scale_kernel(deg2t, h0p):
    return pl.pallas_call(
        _k2_body,
        grid=(8,),
        in_specs=[
            pl.BlockSpec((6400, 2), lambda i: (i, 0)),
            pl.BlockSpec((6400, 32), lambda i: (i, 0)),
        ],
        out_specs=[
            pl.BlockSpec((6400, 1), lambda i: (i, 0)),
            pl.BlockSpec((6400, 32), lambda i: (i, 0)),
        ],
        out_shape=[jax.ShapeDtypeStruct((NPAD, 1), jnp.float32),
                   jax.ShapeDtypeStruct((NPAD, 32), jnp.float32)],
    )(deg2t, h0p)


def _k3_body(agg_ref, hs1_ref, dinv_ref, W1_ref, b1_ref, out_ref):
    i = pl.program_id(0)
    dv = dinv_ref[...]
    t1 = (agg_ref[0] + agg_ref[1] + hs1_ref[...]) * dv
    h1 = jnp.maximum(_mmd(t1, W1_ref[...]) + b1_ref[...], 0.0)
    rows = i * 6400 + lax.broadcasted_iota(jnp.int32, (6400, 1), 0)
    live = lax.rem(rows, S) < 500
    hs2 = jnp.where(live, h1 * dv, 0.0)
    out_ref[0] = hs2[:, 0:32]
    out_ref[1] = hs2[:, 32:64]


def _h1_kernel(agg1, hs1, dinv, W1, b1r):
    return pl.pallas_call(
        _k3_body,
        grid=(8,),
        in_specs=[
            pl.BlockSpec((2, 6400, 32), lambda i: (0, i, 0)),
            pl.BlockSpec((6400, 32), lambda i: (i, 0)),
            pl.BlockSpec((6400, 1), lambda i: (i, 0)),
            pl.BlockSpec((32, 64), lambda i: (0, 0)),
            pl.BlockSpec((1, 64), lambda i: (0, 0)),
        ],
        out_specs=pl.BlockSpec((2, 6400, 32), lambda i: (0, i, 0)),
        out_shape=jax.ShapeDtypeStruct((2, NPAD, 32), jnp.float32),
    )(agg1, hs1, dinv, W1, b1r)


def _k4_body(agg_ref, hs_ref, dinv_ref, W2_ref, b2_ref, fcW_ref, fcb_ref,
             out_ref):
    i = pl.program_id(0)
    dv = dinv_ref[...]
    t2a = (agg_ref[0] + hs_ref[0]) * dv
    t2b = (agg_ref[1] + hs_ref[1]) * dv
    W2 = W2_ref[...]
    h2 = jnp.maximum(
        _mmd(t2a, W2[0:32]) + _mmd(t2b, W2[32:64]) + b2_ref[...], 0.0)
    means = [jnp.sum(h2[b * S:b * S + 500], axis=0, keepdims=True)
             * (1.0 / 500.0) for b in range(4)]
    m = jnp.concatenate(means, axis=0)
    out_ref[pl.ds(i * 4, 4), :] = _mm(m, fcW_ref[...]) + fcb_ref[...]


def _final_kernel(agg2, hs2, dinv, W2, b2r, fc_W, fcb_r):
    return pl.pallas_call(
        _k4_body,
        grid=(25,),
        in_specs=[
            pl.BlockSpec((2, 2048, 32), lambda i: (0, i, 0)),
            pl.BlockSpec((2, 2048, 32), lambda i: (0, i, 0)),
            pl.BlockSpec((2048, 1), lambda i: (i, 0)),
            pl.BlockSpec((64, 64), lambda i: (0, 0)),
            pl.BlockSpec((1, 64), lambda i: (0, 0)),
            pl.BlockSpec((64, 18), lambda i: (0, 0)),
            pl.BlockSpec((1, 18), lambda i: (0, 0)),
        ],
        out_specs=pl.BlockSpec((100, 18), lambda i: (0, 0)),
        out_shape=jax.ShapeDtypeStruct((100, 18), jnp.float32),
    )(agg2, hs2, dinv, W2, b2r, fc_W, fcb_r)


def kernel(x, edge_index, conv1_w, conv1_b, conv2_w, conv2_b,
           W1, b1, W2, b2, fc_W, fc_b):
    f32 = jnp.float32
    xg = jnp.pad(x, ((0, 0), (2, 2), (0, 0))).reshape(100, 501, 36)
    xq = jnp.pad(xg, ((0, 0), (0, S - 501), (0, 0))).reshape(NPAD, 36)
    w1t = jnp.transpose(conv1_w, (2, 1, 0))
    z9 = jnp.zeros((9, 16), f32)
    wA = jnp.concatenate([jnp.concatenate(
        [w1t[o - p] if 0 <= o - p < 5 else z9 for o in range(4)], axis=0)
        for p in (0, 2, 1, 3)], axis=1)
    wB = jnp.concatenate([jnp.concatenate(
        [w1t[o - p] if 0 <= o - p < 5 else z9 for o in range(4, 8)], axis=0)
        for p in (0, 2, 1, 3)], axis=1)
    w2s = jnp.transpose(conv2_w, (2, 1, 0))
    z16 = jnp.zeros((16, 32), f32)
    cat = jnp.concatenate
    wm = cat([cat([w2s[0], z16], 1), cat([w2s[1], w2s[0]], 1)], 0)
    w0 = cat([cat([w2s[2], w2s[1]], 1), cat([w2s[3], w2s[2]], 1)], 0)
    wp = cat([cat([w2s[4], w2s[3]], 1), cat([z16, w2s[4]], 1)], 0)
    b1t = jnp.tile(conv1_b.reshape(1, 16), (1, 4))
    b2t = jnp.tile(conv2_b.reshape(1, 32), (1, 2))

    src = edge_index[0].astype(jnp.int32)
    dst = edge_index[1].astype(jnp.int32)
    src = src + 12 * (src // 500)
    dst = dst + 12 * (dst // 500)
    ji = jnp.arange(EPAD - E, dtype=jnp.int32)
    pad_ids = S * (ji % 100) + 500 + ((ji // 100) % 12)
    srcp = jnp.concatenate([src, pad_ids]).reshape(ECH, 128)
    dstp = jnp.concatenate([dst, pad_ids]).reshape(ECH, 128)
    srcp2 = jnp.stack([srcp, srcp + NPAD])

    gb1 = b1.reshape(1, 64)
    gb2 = b2.reshape(1, 64)
    fcb = fc_b.reshape(1, 18)

    h0 = _conv_stack(xq, wA, wB, wm, w0, wp, b1t, b2t)
    deg2 = _sc_degree(dstp).reshape(2, NPAD)
    deg2t = jnp.transpose(deg2)
    dinv, hs1 = _scale_kernel(deg2t, h0)
    agg1 = _sc_agg(hs1, srcp, dstp, feat_split=False)
    hs2 = _h1_kernel(agg1, hs1, dinv, W1, gb1)
    agg2 = _sc_agg(hs2.reshape(2 * NPAD, 32), srcp2, dstp, feat_split=True)
    return _final_kernel(agg2, hs2, dinv, W2, gb2, fc_W, fcb)
